# Initial kernel scaffold; baseline (speedup 1.0000x reference)
import jax
import jax.numpy as jnp
from jax.experimental import pallas as pl


def kernel(z, pos, edge_index, batch, params):
    raise NotImplementedError("write your pallas kernel here")


# speedup vs baseline: 1.7115x; 1.7115x over previous
import functools

import jax
import jax.numpy as jnp
import numpy as np
from jax import lax
from jax.experimental import pallas as pl
from jax.experimental.pallas import tpu as pltpu
from jax.experimental.pallas import tpu_sc as plsc

N = 10000
E = 160000
H = 128
F = 128
NG = 150
NGP = 160
NI = 3
NUM_EMB = 100
NUM_EMB_P = 128
CUTOFF = 15.0
NG_GRAPHS = 512
LOG2 = float(np.log(2.0))

TILE_N = 1000
TILE_E = 2000

NW = 32
EPW = E // NW
CH = 40
CH2 = 40
NP = 10240
RPT = NP // 16


def _ssp(x):
    return jax.nn.softplus(x) - LOG2


def _f32(shape):
    return jax.ShapeDtypeStruct(shape, jnp.float32)


def _full(shape):
    return pl.BlockSpec(shape, lambda i: (0,) * len(shape))


def _dot(a, b):
    return jnp.dot(a, b, preferred_element_type=jnp.float32)


def _init_body(z_ref, pos_ref, emb_ref, wiT_ref, b1_ref, wjcfT_ref,
               h_ref, pdst_ref, psrc_ref):
    zv = z_ref[...]
    oh = (zv == lax.broadcasted_iota(jnp.int32, (TILE_N, NUM_EMB_P), 1))
    h = _dot(oh.astype(jnp.float32), emb_ref[...])
    h_ref[...] = h
    pp = pos_ref[...]
    pdst_ref[...] = jnp.concatenate(
        [_dot(h, wiT_ref[...]) + b1_ref[...], pp], axis=1)
    psrc_ref[...] = jnp.concatenate([_dot(h, wjcfT_ref[...]), pp], axis=1)


_init_call = pl.pallas_call(
    _init_body,
    grid=(N // TILE_N,),
    in_specs=[
        pl.BlockSpec((TILE_N, 1), lambda i: (i, 0)),
        pl.BlockSpec((TILE_N, H), lambda i: (i, 0)),
        _full((NUM_EMB_P, H)),
        _full((H, 2 * H)),
        _full((1, 2 * H)),
        _full((H, 3 * H)),
    ],
    out_specs=[
        pl.BlockSpec((TILE_N, H), lambda i: (i, 0)),
        pl.BlockSpec((TILE_N, 3 * H), lambda i: (i, 0)),
        pl.BlockSpec((TILE_N, 4 * H), lambda i: (i, 0)),
    ],
    out_shape=[_f32((N, H)), _f32((N, 3 * H)), _f32((N, 4 * H))],
)


def _node_body(msgp_ref, h_ref, sm1T_ref, sm1b_ref, sm2T_ref, sm2b_ref,
               wiT_ref, b1_ref, wjcfT_ref,
               hout_ref, pdst_ref, psrc_ref):
    m = msgp_ref[...]
    msg = m[0] + m[1]
    u = _ssp(_dot(msg, sm1T_ref[...]) + sm1b_ref[...])
    h2 = h_ref[...] + _dot(u, sm2T_ref[...]) + sm2b_ref[...]
    hout_ref[...] = h2
    pdst_ref[...] = _dot(h2, wiT_ref[...]) + b1_ref[...]
    psrc_ref[...] = _dot(h2, wjcfT_ref[...])


_node_call = pl.pallas_call(
    _node_body,
    grid=(N // TILE_N,),
    in_specs=[
        pl.BlockSpec((2, TILE_N, H), lambda i: (0, i, 0)),
        pl.BlockSpec((TILE_N, H), lambda i: (i, 0)),
        _full((H, H)), _full((1, H)), _full((H, H)), _full((1, H)),
        _full((H, 2 * H)), _full((1, 2 * H)), _full((H, 3 * H)),
    ],
    out_specs=[
        pl.BlockSpec((TILE_N, H), lambda i: (i, 0)),
        pl.BlockSpec((TILE_N, 2 * H), lambda i: (i, 0)),
        pl.BlockSpec((TILE_N, 3 * H), lambda i: (i, 0)),
    ],
    out_shape=[_f32((N, H)), _f32((N, 2 * H)), _f32((N, 3 * H))],
)


def _final_body(msgp_ref, h_ref, sm1T_ref, sm1b_ref, sm2T_ref, sm2b_ref,
                l1T_ref, l1b_ref, l2T_ref, l2b_ref, batch_ref, out_ref):
    i = pl.program_id(0)
    m = msgp_ref[...]
    msg = m[0] + m[1]
    u = _ssp(_dot(msg, sm1T_ref[...]) + sm1b_ref[...])
    h2 = h_ref[...] + _dot(u, sm2T_ref[...]) + sm2b_ref[...]
    no = _dot(_ssp(_dot(h2, l1T_ref[...]) + l1b_ref[...]), l2T_ref[...])
    no = no + l2b_ref[...]
    b2 = batch_ref[0]
    mask = (lax.broadcasted_iota(jnp.int32, (NG_GRAPHS, TILE_N), 0) == b2)
    part = _dot(mask.astype(jnp.float32), no)

    @pl.when(i == 0)
    def _zero():
        out_ref[...] = jnp.zeros_like(out_ref)

    out_ref[...] += part


_final_call = pl.pallas_call(
    _final_body,
    grid=(N // TILE_N,),
    in_specs=[
        pl.BlockSpec((2, TILE_N, H), lambda i: (0, i, 0)),
        pl.BlockSpec((TILE_N, H), lambda i: (i, 0)),
        _full((H, H)), _full((1, H)), _full((H, H)), _full((1, H)),
        _full((H, H // 2)), _full((1, H // 2)), _full((H // 2, 1)),
        _full((1, 1)),
        pl.BlockSpec((1, 1, TILE_N), lambda i: (i, 0, 0)),
    ],
    out_specs=pl.BlockSpec((NG_GRAPHS, 1), lambda i: (0, 0)),
    out_shape=_f32((NG_GRAPHS, 1)),
)


_COEFF = -0.5 / (CUTOFF / NG) ** 2


def _edge0_body(gdst_ref, gsrc_ref, offs_ref, weT_ref,
                eu2T_ref, eu2b_ref, f1T_ref, f1b_ref, f2T_ref, f2b_ref,
                eattr_ref, msg_ref):
    gd = gdst_ref[...]
    s = gsrc_ref[...]
    d = gd[:, 2 * H:] - s[:, 3 * H:]
    dist = jnp.sqrt(jnp.sum(d * d, axis=1, keepdims=True))
    g = jnp.exp(_COEFF * (dist - offs_ref[...]) ** 2)
    ea = _ssp(_dot(g, weT_ref[...]) + gd[:, :2 * H] + s[:, :2 * H])
    eattr = _dot(ea, eu2T_ref[...]) + eu2b_ref[...]
    eattr_ref[...] = eattr
    w = _ssp(_dot(eattr, f1T_ref[...]) + f1b_ref[...])
    w = _ssp(_dot(w, f2T_ref[...]) + f2b_ref[...])
    msg_ref[...] = s[:, 2 * H:3 * H] * w


_edge0_call = pl.pallas_call(
    _edge0_body,
    grid=(E // TILE_E,),
    in_specs=[
        pl.BlockSpec((TILE_E, 3 * H), lambda i: (i, 0)),
        pl.BlockSpec((TILE_E, 4 * H), lambda i: (i, 0)),
        _full((1, NGP)), _full((NGP, 2 * H)),
        _full((2 * H, H)), _full((1, H)),
        _full((H, H)), _full((1, H)), _full((H, H)), _full((1, H)),
    ],
    out_specs=[
        pl.BlockSpec((TILE_E, H), lambda i: (i, 0)),
        pl.BlockSpec((TILE_E, H), lambda i: (i, 0)),
    ],
    out_shape=[_f32((E, H)), _f32((E, H))],
)


def _edge1_body(gdst_ref, gsrc_ref, eain_ref, weT_ref,
                eu2T_ref, eu2b_ref, f1T_ref, f1b_ref, f2T_ref, f2b_ref,
                eattr_ref, msg_ref):
    s = gsrc_ref[...]
    ea = _ssp(_dot(eain_ref[...], weT_ref[...]) + gdst_ref[...] + s[:, :2 * H])
    eattr = _dot(ea, eu2T_ref[...]) + eu2b_ref[...]
    eattr_ref[...] = eattr
    w = _ssp(_dot(eattr, f1T_ref[...]) + f1b_ref[...])
    w = _ssp(_dot(w, f2T_ref[...]) + f2b_ref[...])
    msg_ref[...] = s[:, 2 * H:] * w


_edge1_call = pl.pallas_call(
    _edge1_body,
    grid=(E // TILE_E,),
    in_specs=[
        pl.BlockSpec((TILE_E, 2 * H), lambda i: (i, 0)),
        pl.BlockSpec((TILE_E, 3 * H), lambda i: (i, 0)),
        pl.BlockSpec((TILE_E, H), lambda i: (i, 0)),
        _full((H, 2 * H)),
        _full((2 * H, H)), _full((1, H)),
        _full((H, H)), _full((1, H)), _full((H, H)), _full((1, H)),
    ],
    out_specs=[
        pl.BlockSpec((TILE_E, H), lambda i: (i, 0)),
        pl.BlockSpec((TILE_E, H), lambda i: (i, 0)),
    ],
    out_shape=[_f32((E, H)), _f32((E, H))],
)


def _edge2_body(gdst_ref, gsrc_ref, eain_ref, weT_ref,
                wcT_ref, bc_ref, f2T_ref, f2b_ref, msg_ref):
    s = gsrc_ref[...]
    ea = _ssp(_dot(eain_ref[...], weT_ref[...]) + gdst_ref[...] + s[:, :2 * H])
    w = _ssp(_dot(ea, wcT_ref[...]) + bc_ref[...])
    w = _ssp(_dot(w, f2T_ref[...]) + f2b_ref[...])
    msg_ref[...] = s[:, 2 * H:] * w


_edge2_call = pl.pallas_call(
    _edge2_body,
    grid=(E // TILE_E,),
    in_specs=[
        pl.BlockSpec((TILE_E, 2 * H), lambda i: (i, 0)),
        pl.BlockSpec((TILE_E, 3 * H), lambda i: (i, 0)),
        pl.BlockSpec((TILE_E, H), lambda i: (i, 0)),
        _full((H, 2 * H)),
        _full((2 * H, H)), _full((1, H)),
        _full((H, H)), _full((1, H)),
    ],
    out_specs=pl.BlockSpec((TILE_E, H), lambda i: (i, 0)),
    out_shape=_f32((E, H)),
)


_MESH = plsc.VectorSubcoreMesh(core_axis_name="c", subcore_axis_name="s")


def _make_gather2(d1, d2):
    @functools.partial(
        pl.kernel,
        out_type=(_f32((E, d1)), _f32((E, d2))),
        mesh=_MESH,
        scratch_types=[
            pltpu.VMEM((CH,), jnp.int32),
            pltpu.VMEM((CH,), jnp.int32),
            pltpu.VMEM((CH, d1), jnp.float32),
            pltpu.VMEM((CH, d2), jnp.float32),
            pltpu.SemaphoreType.DMA,
            pltpu.SemaphoreType.DMA,
        ],
    )
    def gath(t1_hbm, i1_hbm, t2_hbm, i2_hbm, o1_hbm, o2_hbm,
             idx1, idx2, r1, r2, s1, s2):
        wid = lax.axis_index("s") * 2 + lax.axis_index("c")
        base = wid * EPW

        def body(ci, carry):
            off = base + ci * CH
            pltpu.sync_copy(i1_hbm.at[pl.ds(off, CH)], idx1)
            pltpu.sync_copy(i2_hbm.at[pl.ds(off, CH)], idx2)
            c1 = pltpu.async_copy(t1_hbm.at[idx1], r1, s1)
            c2 = pltpu.async_copy(t2_hbm.at[idx2], r2, s2)
            c1.wait()
            c2.wait()
            pltpu.sync_copy(r1, o1_hbm.at[pl.ds(off, CH)])
            pltpu.sync_copy(r2, o2_hbm.at[pl.ds(off, CH)])
            return carry

        lax.fori_loop(0, EPW // CH, body, 0)

    return gath


_gather_tables = _make_gather2(2 * H, 3 * H)
_gather_tables0 = _make_gather2(3 * H, 4 * H)


@functools.partial(
    pl.kernel,
    out_type=_f32((2, NP, H)),
    mesh=_MESH,
    scratch_types=[
        pltpu.VMEM((CH2,), jnp.int32),
        pltpu.VMEM((CH2, H), jnp.float32),
        pltpu.VMEM_SHARED((NP, H), jnp.float32),
    ],
)
def _scatter_call(msg_hbm, dst_hbm, zer_hbm, out_hbm, idx, rows, acc):
    c = lax.axis_index("c")
    s = lax.axis_index("s")
    wid = s * 2 + c
    pltpu.sync_copy(zer_hbm, acc.at[pl.ds(s * RPT, RPT)])
    plsc.subcore_barrier()
    base = wid * EPW

    def body(ci, carry):
        off = base + ci * CH2
        pltpu.sync_copy(dst_hbm.at[pl.ds(off, CH2)], idx)
        pltpu.sync_copy(msg_hbm.at[pl.ds(off, CH2)], rows)
        pltpu.sync_copy(rows, acc.at[idx], add=True)
        return carry

    lax.fori_loop(0, EPW // CH2, body, 0)
    plsc.subcore_barrier()
    pltpu.sync_copy(acc.at[pl.ds(s * RPT, RPT)],
                    out_hbm.at[c, pl.ds(s * RPT, RPT)])


def kernel(z, pos, edge_index, batch, params):
    src = edge_index[0].astype(jnp.int32)
    dst = edge_index[1].astype(jnp.int32)
    z2 = z.astype(jnp.int32).reshape(N, 1)
    batch2 = batch.astype(jnp.int32).reshape(N // TILE_N, 1, TILE_N)
    pospad = jnp.pad(pos.astype(jnp.float32), ((0, 0), (0, H - 3)))
    zer = jnp.zeros((RPT, H), jnp.float32)

    stop = CUTOFF - CUTOFF / NG
    offs = jnp.pad(jnp.linspace(0.0, stop, NG, dtype=jnp.float32),
                   (0, NGP - NG)).reshape(1, NGP)

    emb = jnp.pad(params['embedding'], ((0, NUM_EMB_P - NUM_EMB), (0, 0)))

    prep = []
    for t, p in enumerate(params['interactions']):
        w1 = p['eu1_W']
        wiT = w1[:, :H].T
        wjT = w1[:, H:2 * H].T
        weT = w1[:, 2 * H:].T
        if t == 0:
            weT = jnp.pad(weT, ((0, NGP - NG), (0, 0)))
        wjcfT = jnp.concatenate([wjT, p['cf_W'].T], axis=1)
        d = dict(
            wiT=wiT, b1=p['eu1_b'].reshape(1, 2 * H), wjcfT=wjcfT, weT=weT,
            eu2T=p['eu2_W'].T, eu2b=p['eu2_b'].reshape(1, H),
            f1T=p['f1_W'].T, f1b=p['f1_b'].reshape(1, H),
            f2T=p['f2_W'].T, f2b=p['f2_b'].reshape(1, H),
            sm1T=p['sm1_W'].T, sm1b=p['sm1_b'].reshape(1, H),
            sm2T=p['sm2_W'].T, sm2b=p['sm2_b'].reshape(1, H),
        )
        if t == NI - 1:
            d['wcT'] = p['eu2_W'].T @ p['f1_W'].T
            d['bc'] = (p['eu2_b'] @ p['f1_W'].T
                       + p['f1_b']).reshape(1, H)
        prep.append(d)

    p0, p1, p2 = prep

    h0, pd, ps = _init_call(z2, pospad, emb, p0['wiT'], p0['b1'], p0['wjcfT'])

    gd, gs = _gather_tables0(pd, dst, ps, src)
    eattr, msg_e = _edge0_call(gd, gs, offs, p0['weT'],
                               p0['eu2T'], p0['eu2b'], p0['f1T'], p0['f1b'],
                               p0['f2T'], p0['f2b'])
    msgp = _scatter_call(msg_e, dst, zer)
    h1, pd, ps = _node_call(msgp, h0, p0['sm1T'], p0['sm1b'], p0['sm2T'],
                            p0['sm2b'], p1['wiT'], p1['b1'], p1['wjcfT'])

    gd, gs = _gather_tables(pd, dst, ps, src)
    eattr, msg_e = _edge1_call(gd, gs, eattr, p1['weT'],
                               p1['eu2T'], p1['eu2b'], p1['f1T'], p1['f1b'],
                               p1['f2T'], p1['f2b'])
    msgp = _scatter_call(msg_e, dst, zer)
    h2, pd, ps = _node_call(msgp, h1, p1['sm1T'], p1['sm1b'], p1['sm2T'],
                            p1['sm2b'], p2['wiT'], p2['b1'], p2['wjcfT'])

    gd, gs = _gather_tables(pd, dst, ps, src)
    msg_e = _edge2_call(gd, gs, eattr, p2['weT'],
                        p2['wcT'], p2['bc'], p2['f2T'], p2['f2b'])
    msgp = _scatter_call(msg_e, dst, zer)

    lin1T = params['lin1_W'].T
    lin2T = params['lin2_W'].T
    out = _final_call(msgp, h2, p2['sm1T'], p2['sm1b'], p2['sm2T'],
                      p2['sm2b'], lin1T,
                      params['lin1_b'].reshape(1, H // 2), lin2T,
                      params['lin2_b'].reshape(1, 1), batch2)
    return out


# speedup vs baseline: 2.2220x; 1.2983x over previous
import functools

import jax
import jax.numpy as jnp
import numpy as np
from jax import lax
from jax.experimental import pallas as pl
from jax.experimental.pallas import tpu as pltpu
from jax.experimental.pallas import tpu_sc as plsc

N = 10000
E = 160000
H = 128
F = 128
NG = 150
NGP = 160
NI = 3
NUM_EMB = 100
NUM_EMB_P = 128
CUTOFF = 15.0
NG_GRAPHS = 512
LOG2 = float(np.log(2.0))

TILE_N = 1000
TILE_E = 2000

NW = 32
EPW = E // NW
CH = 40
CH2 = 40
NP = 10240
RPT = NP // 16


def _ssp(x):
    return jax.nn.softplus(x) - LOG2


def _f32(shape):
    return jax.ShapeDtypeStruct(shape, jnp.float32)


def _full(shape):
    return pl.BlockSpec(shape, lambda i: (0,) * len(shape))


def _dot(a, b):
    return jnp.dot(a, b, preferred_element_type=jnp.float32)


def _init_body(z_ref, pos_ref, emb_ref, wiT_ref, b1_ref, wjcfT_ref,
               h_ref, pdst_ref, psrc_ref):
    zv = z_ref[...]
    oh = (zv == lax.broadcasted_iota(jnp.int32, (TILE_N, NUM_EMB_P), 1))
    h = _dot(oh.astype(jnp.float32), emb_ref[...])
    h_ref[...] = h
    pp = pos_ref[...]
    pdst_ref[...] = jnp.concatenate(
        [_dot(h, wiT_ref[...]) + b1_ref[...], pp], axis=1)
    psrc_ref[...] = jnp.concatenate([_dot(h, wjcfT_ref[...]), pp], axis=1)


_init_call = pl.pallas_call(
    _init_body,
    grid=(N // TILE_N,),
    in_specs=[
        pl.BlockSpec((TILE_N, 1), lambda i: (i, 0)),
        pl.BlockSpec((TILE_N, H), lambda i: (i, 0)),
        _full((NUM_EMB_P, H)),
        _full((H, 2 * H)),
        _full((1, 2 * H)),
        _full((H, 3 * H)),
    ],
    out_specs=[
        pl.BlockSpec((TILE_N, H), lambda i: (i, 0)),
        pl.BlockSpec((TILE_N, 3 * H), lambda i: (i, 0)),
        pl.BlockSpec((TILE_N, 4 * H), lambda i: (i, 0)),
    ],
    out_shape=[_f32((N, H)), _f32((N, 3 * H)), _f32((N, 4 * H))],
)


def _node_body(msgp_ref, h_ref, sm1T_ref, sm1b_ref, sm2T_ref, sm2b_ref,
               wiT_ref, b1_ref, wjcfT_ref,
               hout_ref, pdst_ref, psrc_ref):
    m = msgp_ref[...]
    msg = m[0] + m[1]
    u = _ssp(_dot(msg, sm1T_ref[...]) + sm1b_ref[...])
    h2 = h_ref[...] + _dot(u, sm2T_ref[...]) + sm2b_ref[...]
    hout_ref[...] = h2
    pdst_ref[...] = _dot(h2, wiT_ref[...]) + b1_ref[...]
    psrc_ref[...] = _dot(h2, wjcfT_ref[...])


_node_call = pl.pallas_call(
    _node_body,
    grid=(N // TILE_N,),
    in_specs=[
        pl.BlockSpec((2, TILE_N, H), lambda i: (0, i, 0)),
        pl.BlockSpec((TILE_N, H), lambda i: (i, 0)),
        _full((H, H)), _full((1, H)), _full((H, H)), _full((1, H)),
        _full((H, 2 * H)), _full((1, 2 * H)), _full((H, 3 * H)),
    ],
    out_specs=[
        pl.BlockSpec((TILE_N, H), lambda i: (i, 0)),
        pl.BlockSpec((TILE_N, 2 * H), lambda i: (i, 0)),
        pl.BlockSpec((TILE_N, 3 * H), lambda i: (i, 0)),
    ],
    out_shape=[_f32((N, H)), _f32((N, 2 * H)), _f32((N, 3 * H))],
)


def _final_body(msgp_ref, h_ref, sm1T_ref, sm1b_ref, sm2T_ref, sm2b_ref,
                l1T_ref, l1b_ref, l2T_ref, l2b_ref, batch_ref, out_ref):
    i = pl.program_id(0)
    m = msgp_ref[...]
    msg = m[0] + m[1]
    u = _ssp(_dot(msg, sm1T_ref[...]) + sm1b_ref[...])
    h2 = h_ref[...] + _dot(u, sm2T_ref[...]) + sm2b_ref[...]
    no = _dot(_ssp(_dot(h2, l1T_ref[...]) + l1b_ref[...]), l2T_ref[...])
    no = no + l2b_ref[...]
    b2 = batch_ref[0]
    mask = (lax.broadcasted_iota(jnp.int32, (NG_GRAPHS, TILE_N), 0) == b2)
    part = _dot(mask.astype(jnp.float32), no)

    @pl.when(i == 0)
    def _zero():
        out_ref[...] = jnp.zeros_like(out_ref)

    out_ref[...] += part


_final_call = pl.pallas_call(
    _final_body,
    grid=(N // TILE_N,),
    in_specs=[
        pl.BlockSpec((2, TILE_N, H), lambda i: (0, i, 0)),
        pl.BlockSpec((TILE_N, H), lambda i: (i, 0)),
        _full((H, H)), _full((1, H)), _full((H, H)), _full((1, H)),
        _full((H, H // 2)), _full((1, H // 2)), _full((H // 2, 1)),
        _full((1, 1)),
        pl.BlockSpec((1, 1, TILE_N), lambda i: (i, 0, 0)),
    ],
    out_specs=pl.BlockSpec((NG_GRAPHS, 1), lambda i: (0, 0)),
    out_shape=_f32((NG_GRAPHS, 1)),
)


_COEFF = -0.5 / (CUTOFF / NG) ** 2


def _edge0_body(gdst_ref, gsrc_ref, offs_ref, weT_ref,
                eu2T_ref, eu2b_ref, f1T_ref, f1b_ref, f2T_ref, f2b_ref,
                eattr_ref, msg_ref):
    gd = gdst_ref[...]
    s = gsrc_ref[...]
    d = gd[:, 2 * H:] - s[:, 3 * H:]
    dist = jnp.sqrt(jnp.sum(d * d, axis=1, keepdims=True))
    g = jnp.exp(_COEFF * (dist - offs_ref[...]) ** 2)
    ea = _ssp(_dot(g, weT_ref[...]) + gd[:, :2 * H] + s[:, :2 * H])
    eattr = _dot(ea, eu2T_ref[...]) + eu2b_ref[...]
    eattr_ref[...] = eattr
    w = _ssp(_dot(eattr, f1T_ref[...]) + f1b_ref[...])
    w = _ssp(_dot(w, f2T_ref[...]) + f2b_ref[...])
    msg_ref[...] = s[:, 2 * H:3 * H] * w


_edge0_call = pl.pallas_call(
    _edge0_body,
    grid=(E // TILE_E,),
    in_specs=[
        pl.BlockSpec((TILE_E, 3 * H), lambda i: (i, 0)),
        pl.BlockSpec((TILE_E, 4 * H), lambda i: (i, 0)),
        _full((1, NGP)), _full((NGP, 2 * H)),
        _full((2 * H, H)), _full((1, H)),
        _full((H, H)), _full((1, H)), _full((H, H)), _full((1, H)),
    ],
    out_specs=[
        pl.BlockSpec((TILE_E, H), lambda i: (i, 0)),
        pl.BlockSpec((TILE_E, H), lambda i: (i, 0)),
    ],
    out_shape=[_f32((E, H)), _f32((E, H))],
)


def _edge1_body(gdst_ref, gsrc_ref, eain_ref, weT_ref,
                eu2T_ref, eu2b_ref, f1T_ref, f1b_ref, f2T_ref, f2b_ref,
                eattr_ref, msg_ref):
    s = gsrc_ref[...]
    ea = _ssp(_dot(eain_ref[...], weT_ref[...]) + gdst_ref[...] + s[:, :2 * H])
    eattr = _dot(ea, eu2T_ref[...]) + eu2b_ref[...]
    eattr_ref[...] = eattr
    w = _ssp(_dot(eattr, f1T_ref[...]) + f1b_ref[...])
    w = _ssp(_dot(w, f2T_ref[...]) + f2b_ref[...])
    msg_ref[...] = s[:, 2 * H:] * w


_edge1_call = pl.pallas_call(
    _edge1_body,
    grid=(E // TILE_E,),
    in_specs=[
        pl.BlockSpec((TILE_E, 2 * H), lambda i: (i, 0)),
        pl.BlockSpec((TILE_E, 3 * H), lambda i: (i, 0)),
        pl.BlockSpec((TILE_E, H), lambda i: (i, 0)),
        _full((H, 2 * H)),
        _full((2 * H, H)), _full((1, H)),
        _full((H, H)), _full((1, H)), _full((H, H)), _full((1, H)),
    ],
    out_specs=[
        pl.BlockSpec((TILE_E, H), lambda i: (i, 0)),
        pl.BlockSpec((TILE_E, H), lambda i: (i, 0)),
    ],
    out_shape=[_f32((E, H)), _f32((E, H))],
)


def _edge2_body(gdst_ref, gsrc_ref, eain_ref, weT_ref,
                wcT_ref, bc_ref, f2T_ref, f2b_ref, msg_ref):
    s = gsrc_ref[...]
    ea = _ssp(_dot(eain_ref[...], weT_ref[...]) + gdst_ref[...] + s[:, :2 * H])
    w = _ssp(_dot(ea, wcT_ref[...]) + bc_ref[...])
    w = _ssp(_dot(w, f2T_ref[...]) + f2b_ref[...])
    msg_ref[...] = s[:, 2 * H:] * w


_edge2_call = pl.pallas_call(
    _edge2_body,
    grid=(E // TILE_E,),
    in_specs=[
        pl.BlockSpec((TILE_E, 2 * H), lambda i: (i, 0)),
        pl.BlockSpec((TILE_E, 3 * H), lambda i: (i, 0)),
        pl.BlockSpec((TILE_E, H), lambda i: (i, 0)),
        _full((H, 2 * H)),
        _full((2 * H, H)), _full((1, H)),
        _full((H, H)), _full((1, H)),
    ],
    out_specs=pl.BlockSpec((TILE_E, H), lambda i: (i, 0)),
    out_shape=_f32((E, H)),
)


_MESH = plsc.VectorSubcoreMesh(core_axis_name="c", subcore_axis_name="s")


def _make_gather2(d1, d2):
    nch = EPW // CH
    assert nch % 2 == 1

    @functools.partial(
        pl.kernel,
        out_type=(_f32((E, d1)), _f32((E, d2))),
        mesh=_MESH,
        scratch_types=[
            pltpu.VMEM((CH,), jnp.int32), pltpu.VMEM((CH,), jnp.int32),
            pltpu.VMEM((CH,), jnp.int32), pltpu.VMEM((CH,), jnp.int32),
            pltpu.VMEM((CH, d1), jnp.float32),
            pltpu.VMEM((CH, d1), jnp.float32),
            pltpu.VMEM((CH, d2), jnp.float32),
            pltpu.VMEM((CH, d2), jnp.float32),
            pltpu.SemaphoreType.DMA, pltpu.SemaphoreType.DMA,
            pltpu.SemaphoreType.DMA, pltpu.SemaphoreType.DMA,
        ],
    )
    def gath(t1_hbm, i1_hbm, t2_hbm, i2_hbm, o1_hbm, o2_hbm,
             ia0, ia1, ib0, ib1, ra0, ra1, rb0, rb1, sa0, sa1, sb0, sb1):
        ia, ib = (ia0, ia1), (ib0, ib1)
        ra, rb = (ra0, ra1), (rb0, rb1)
        sa, sb = (sa0, sa1), (sb0, sb1)
        wid = lax.axis_index("s") * 2 + lax.axis_index("c")
        base = wid * EPW

        def load_start(buf, off):
            pltpu.sync_copy(i1_hbm.at[pl.ds(off, CH)], ia[buf])
            pltpu.sync_copy(i2_hbm.at[pl.ds(off, CH)], ib[buf])
            pltpu.async_copy(t1_hbm.at[ia[buf]], ra[buf], sa[buf])
            pltpu.async_copy(t2_hbm.at[ib[buf]], rb[buf], sb[buf])

        def wait_store(buf, off):
            pltpu.make_async_copy(t1_hbm.at[ia[buf]], ra[buf], sa[buf]).wait()
            pltpu.make_async_copy(t2_hbm.at[ib[buf]], rb[buf], sb[buf]).wait()
            pltpu.sync_copy(ra[buf], o1_hbm.at[pl.ds(off, CH)])
            pltpu.sync_copy(rb[buf], o2_hbm.at[pl.ds(off, CH)])

        load_start(0, base)

        @pl.loop(0, nch - 1, step=2)
        def _pair(ci):
            for b in range(2):
                off = base + (ci + b) * CH
                load_start(1 - b, off + CH)
                wait_store(b, off)

        wait_store(0, base + (nch - 1) * CH)

    return gath


_gather_tables = _make_gather2(2 * H, 3 * H)
_gather_tables0 = _make_gather2(3 * H, 4 * H)


_NCH2 = EPW // CH2
assert _NCH2 % 2 == 1


@functools.partial(
    pl.kernel,
    out_type=_f32((2, NP, H)),
    mesh=_MESH,
    scratch_types=[
        pltpu.VMEM((CH2,), jnp.int32), pltpu.VMEM((CH2,), jnp.int32),
        pltpu.VMEM((CH2, H), jnp.float32), pltpu.VMEM((CH2, H), jnp.float32),
        pltpu.VMEM_SHARED((NP, H), jnp.float32),
        pltpu.SemaphoreType.DMA, pltpu.SemaphoreType.DMA,
    ],
)
def _scatter_call(msg_hbm, dst_hbm, zer_hbm, out_hbm,
                  i0, i1, r0, r1, acc, s0, s1):
    c = lax.axis_index("c")
    s = lax.axis_index("s")
    wid = s * 2 + c
    idx, rows, sem = (i0, i1), (r0, r1), (s0, s1)
    pltpu.sync_copy(zer_hbm, acc.at[pl.ds(s * RPT, RPT)])
    plsc.subcore_barrier()
    base = wid * EPW

    def load_start(buf, off):
        pltpu.sync_copy(dst_hbm.at[pl.ds(off, CH2)], idx[buf])
        pltpu.async_copy(msg_hbm.at[pl.ds(off, CH2)], rows[buf], sem[buf])

    def wait_add(buf, off):
        pltpu.make_async_copy(msg_hbm.at[pl.ds(off, CH2)], rows[buf],
                              sem[buf]).wait()
        pltpu.sync_copy(rows[buf], acc.at[idx[buf]], add=True)

    load_start(0, base)

    @pl.loop(0, _NCH2 - 1, step=2)
    def _pair(ci):
        for b in range(2):
            off = base + (ci + b) * CH2
            load_start(1 - b, off + CH2)
            wait_add(b, off)

    wait_add(0, base + (_NCH2 - 1) * CH2)
    plsc.subcore_barrier()
    pltpu.sync_copy(acc.at[pl.ds(s * RPT, RPT)],
                    out_hbm.at[c, pl.ds(s * RPT, RPT)])


def kernel(z, pos, edge_index, batch, params):
    src = edge_index[0].astype(jnp.int32)
    dst = edge_index[1].astype(jnp.int32)
    z2 = z.astype(jnp.int32).reshape(N, 1)
    batch2 = batch.astype(jnp.int32).reshape(N // TILE_N, 1, TILE_N)
    pospad = jnp.pad(pos.astype(jnp.float32), ((0, 0), (0, H - 3)))
    zer = jnp.zeros((RPT, H), jnp.float32)

    stop = CUTOFF - CUTOFF / NG
    offs = jnp.pad(jnp.linspace(0.0, stop, NG, dtype=jnp.float32),
                   (0, NGP - NG)).reshape(1, NGP)

    emb = jnp.pad(params['embedding'], ((0, NUM_EMB_P - NUM_EMB), (0, 0)))

    prep = []
    for t, p in enumerate(params['interactions']):
        w1 = p['eu1_W']
        wiT = w1[:, :H].T
        wjT = w1[:, H:2 * H].T
        weT = w1[:, 2 * H:].T
        if t == 0:
            weT = jnp.pad(weT, ((0, NGP - NG), (0, 0)))
        wjcfT = jnp.concatenate([wjT, p['cf_W'].T], axis=1)
        d = dict(
            wiT=wiT, b1=p['eu1_b'].reshape(1, 2 * H), wjcfT=wjcfT, weT=weT,
            eu2T=p['eu2_W'].T, eu2b=p['eu2_b'].reshape(1, H),
            f1T=p['f1_W'].T, f1b=p['f1_b'].reshape(1, H),
            f2T=p['f2_W'].T, f2b=p['f2_b'].reshape(1, H),
            sm1T=p['sm1_W'].T, sm1b=p['sm1_b'].reshape(1, H),
            sm2T=p['sm2_W'].T, sm2b=p['sm2_b'].reshape(1, H),
        )
        if t == NI - 1:
            d['wcT'] = p['eu2_W'].T @ p['f1_W'].T
            d['bc'] = (p['eu2_b'] @ p['f1_W'].T
                       + p['f1_b']).reshape(1, H)
        prep.append(d)

    p0, p1, p2 = prep

    h0, pd, ps = _init_call(z2, pospad, emb, p0['wiT'], p0['b1'], p0['wjcfT'])

    gd, gs = _gather_tables0(pd, dst, ps, src)
    eattr, msg_e = _edge0_call(gd, gs, offs, p0['weT'],
                               p0['eu2T'], p0['eu2b'], p0['f1T'], p0['f1b'],
                               p0['f2T'], p0['f2b'])
    msgp = _scatter_call(msg_e, dst, zer)
    h1, pd, ps = _node_call(msgp, h0, p0['sm1T'], p0['sm1b'], p0['sm2T'],
                            p0['sm2b'], p1['wiT'], p1['b1'], p1['wjcfT'])

    gd, gs = _gather_tables(pd, dst, ps, src)
    eattr, msg_e = _edge1_call(gd, gs, eattr, p1['weT'],
                               p1['eu2T'], p1['eu2b'], p1['f1T'], p1['f1b'],
                               p1['f2T'], p1['f2b'])
    msgp = _scatter_call(msg_e, dst, zer)
    h2, pd, ps = _node_call(msgp, h1, p1['sm1T'], p1['sm1b'], p1['sm2T'],
                            p1['sm2b'], p2['wiT'], p2['b1'], p2['wjcfT'])

    gd, gs = _gather_tables(pd, dst, ps, src)
    msg_e = _edge2_call(gd, gs, eattr, p2['weT'],
                        p2['wcT'], p2['bc'], p2['f2T'], p2['f2b'])
    msgp = _scatter_call(msg_e, dst, zer)

    lin1T = params['lin1_W'].T
    lin2T = params['lin2_W'].T
    out = _final_call(msgp, h2, p2['sm1T'], p2['sm1b'], p2['sm2T'],
                      p2['sm2b'], lin1T,
                      params['lin1_b'].reshape(1, H // 2), lin2T,
                      params['lin2_b'].reshape(1, 1), batch2)
    return out


# speedup vs baseline: 2.4726x; 1.1128x over previous
import functools

import jax
import jax.numpy as jnp
import numpy as np
from jax import lax
from jax.experimental import pallas as pl
from jax.experimental.pallas import tpu as pltpu
from jax.experimental.pallas import tpu_sc as plsc

N = 10000
E = 160000
H = 128
F = 128
NG = 150
NGP = 160
NI = 3
NUM_EMB = 100
NUM_EMB_P = 128
CUTOFF = 15.0
NG_GRAPHS = 512
LOG2 = float(np.log(2.0))

TILE_N = 1000
TILE_E = 2000

NW = 32
EPW = E // NW
CH = 40
CH2 = 40
NP = 10240
RPT = NP // 16


def _ssp(x):
    return jax.nn.softplus(x) - LOG2


def _f32(shape):
    return jax.ShapeDtypeStruct(shape, jnp.float32)


def _bf16(shape):
    return jax.ShapeDtypeStruct(shape, jnp.bfloat16)


_MASKHI = -65536


def _pack2(a, b):
    ab = lax.bitcast_convert_type(
        a.astype(jnp.bfloat16).astype(jnp.float32), jnp.int32)
    bb = lax.bitcast_convert_type(
        b.astype(jnp.bfloat16).astype(jnp.float32), jnp.int32)
    word = jnp.bitwise_or(lax.shift_right_logical(ab, 16),
                          jnp.bitwise_and(bb, _MASKHI))
    return lax.bitcast_convert_type(word, jnp.float32)


def _unpack2(p):
    u = lax.bitcast_convert_type(p, jnp.int32)
    a = lax.bitcast_convert_type(lax.shift_left(u, 16), jnp.float32)
    b = lax.bitcast_convert_type(jnp.bitwise_and(u, _MASKHI), jnp.float32)
    return a, b


def _unpack_cat(p):
    a, b = _unpack2(p)
    return jnp.concatenate([a, b], axis=1)


def _full(shape):
    return pl.BlockSpec(shape, lambda i: (0,) * len(shape))


def _dot(a, b):
    return jnp.dot(a, b, preferred_element_type=jnp.float32)


def _bdot(a, b_ref):
    return jnp.dot(a.astype(jnp.bfloat16), b_ref[...],
                   preferred_element_type=jnp.float32)


def _init_body(z_ref, pos_ref, emb_ref, wiT_ref, b1_ref, wjT_ref, cfT_ref,
               h_ref, pdst_ref, psrc_ref):
    zv = z_ref[...]
    oh = (zv == lax.broadcasted_iota(jnp.int32, (TILE_N, NUM_EMB_P), 1))
    h = _dot(oh.astype(jnp.float32), emb_ref[...])
    h_ref[...] = h
    pp = pos_ref[...]
    vi = _dot(h, wiT_ref[...]) + b1_ref[...]
    vj = _dot(h, wjT_ref[...])
    cf = _dot(h, cfT_ref[...])
    pdst_ref[...] = jnp.concatenate(
        [_pack2(vi[:, :H], vi[:, H:]), pp], axis=1)
    psrc_ref[...] = jnp.concatenate(
        [_pack2(vj[:, :H], vj[:, H:]),
         _pack2(cf[:, :H // 2], cf[:, H // 2:]),
         pp[:, :H // 2]], axis=1)


_init_call = pl.pallas_call(
    _init_body,
    grid=(N // TILE_N,),
    in_specs=[
        pl.BlockSpec((TILE_N, 1), lambda i: (i, 0)),
        pl.BlockSpec((TILE_N, H), lambda i: (i, 0)),
        _full((NUM_EMB_P, H)),
        _full((H, 2 * H)),
        _full((1, 2 * H)),
        _full((H, 2 * H)),
        _full((H, H)),
    ],
    out_specs=[
        pl.BlockSpec((TILE_N, H), lambda i: (i, 0)),
        pl.BlockSpec((TILE_N, 2 * H), lambda i: (i, 0)),
        pl.BlockSpec((TILE_N, 2 * H), lambda i: (i, 0)),
    ],
    out_shape=[_f32((N, H)), _f32((N, 2 * H)), _f32((N, 2 * H))],
)


def _node_body(msgp_ref, h_ref, sm1T_ref, sm1b_ref, sm2T_ref, sm2b_ref,
               wiT_ref, b1_ref, wjT_ref, cfT_ref,
               hout_ref, pdst_ref, psrc_ref):
    m = msgp_ref[...]
    msg = m[0] + m[1]
    u = _ssp(_dot(msg, sm1T_ref[...]) + sm1b_ref[...])
    h2 = h_ref[...] + _dot(u, sm2T_ref[...]) + sm2b_ref[...]
    hout_ref[...] = h2
    vi = _dot(h2, wiT_ref[...]) + b1_ref[...]
    vj = _dot(h2, wjT_ref[...])
    cf = _dot(h2, cfT_ref[...])
    pdst_ref[...] = _pack2(vi[:, :H], vi[:, H:])
    psrc_ref[...] = jnp.concatenate([_pack2(vj[:, :H], vj[:, H:]), cf],
                                    axis=1)


_node_call = pl.pallas_call(
    _node_body,
    grid=(N // TILE_N,),
    in_specs=[
        pl.BlockSpec((2, TILE_N, H), lambda i: (0, i, 0)),
        pl.BlockSpec((TILE_N, H), lambda i: (i, 0)),
        _full((H, H)), _full((1, H)), _full((H, H)), _full((1, H)),
        _full((H, 2 * H)), _full((1, 2 * H)), _full((H, 2 * H)),
        _full((H, H)),
    ],
    out_specs=[
        pl.BlockSpec((TILE_N, H), lambda i: (i, 0)),
        pl.BlockSpec((TILE_N, H), lambda i: (i, 0)),
        pl.BlockSpec((TILE_N, 2 * H), lambda i: (i, 0)),
    ],
    out_shape=[_f32((N, H)), _f32((N, H)), _f32((N, 2 * H))],
)


def _final_body(msgp_ref, h_ref, sm1T_ref, sm1b_ref, sm2T_ref, sm2b_ref,
                l1T_ref, l1b_ref, l2T_ref, l2b_ref, batch_ref, out_ref):
    i = pl.program_id(0)
    m = msgp_ref[...]
    msg = m[0] + m[1]
    u = _ssp(_dot(msg, sm1T_ref[...]) + sm1b_ref[...])
    h2 = h_ref[...] + _dot(u, sm2T_ref[...]) + sm2b_ref[...]
    no = _dot(_ssp(_dot(h2, l1T_ref[...]) + l1b_ref[...]), l2T_ref[...])
    no = no + l2b_ref[...]
    b2 = batch_ref[0]
    mask = (lax.broadcasted_iota(jnp.int32, (NG_GRAPHS, TILE_N), 0) == b2)
    part = _dot(mask.astype(jnp.float32), no)

    @pl.when(i == 0)
    def _zero():
        out_ref[...] = jnp.zeros_like(out_ref)

    out_ref[...] += part


_final_call = pl.pallas_call(
    _final_body,
    grid=(N // TILE_N,),
    in_specs=[
        pl.BlockSpec((2, TILE_N, H), lambda i: (0, i, 0)),
        pl.BlockSpec((TILE_N, H), lambda i: (i, 0)),
        _full((H, H)), _full((1, H)), _full((H, H)), _full((1, H)),
        _full((H, H // 2)), _full((1, H // 2)), _full((H // 2, 1)),
        _full((1, 1)),
        pl.BlockSpec((1, 1, TILE_N), lambda i: (i, 0, 0)),
    ],
    out_specs=pl.BlockSpec((NG_GRAPHS, 1), lambda i: (0, 0)),
    out_shape=_f32((NG_GRAPHS, 1)),
)


_COEFF = -0.5 / (CUTOFF / NG) ** 2


def _edge0_body(gdst_ref, gsrc_ref, offs_ref, weT_ref,
                eu2T_ref, eu2b_ref, f1T_ref, f1b_ref, f2T_ref, f2b_ref,
                eattr_ref, msg_ref):
    gd = gdst_ref[...]
    s = gsrc_ref[...]
    d = gd[:, H:H + H // 2] - s[:, H + H // 2:]
    dist = jnp.sqrt(jnp.sum(d * d, axis=1, keepdims=True))
    g = jnp.exp(_COEFF * (dist - offs_ref[...]) ** 2)
    pd = _unpack_cat(gd[:, :H])
    pj = _unpack_cat(s[:, :H])
    cf = _unpack_cat(s[:, H:H + H // 2])
    ea = _ssp(_bdot(g, weT_ref) + pd + pj)
    eattr = _bdot(ea, eu2T_ref) + eu2b_ref[...]
    eattr_ref[...] = eattr.astype(jnp.bfloat16)
    w = _ssp(_bdot(eattr, f1T_ref) + f1b_ref[...])
    w = _ssp(_bdot(w, f2T_ref) + f2b_ref[...])
    msg_ref[...] = cf * w


_edge0_call = pl.pallas_call(
    _edge0_body,
    grid=(E // TILE_E,),
    in_specs=[
        pl.BlockSpec((TILE_E, 2 * H), lambda i: (i, 0)),
        pl.BlockSpec((TILE_E, 2 * H), lambda i: (i, 0)),
        _full((1, NGP)), _full((NGP, 2 * H)),
        _full((2 * H, H)), _full((1, H)),
        _full((H, H)), _full((1, H)), _full((H, H)), _full((1, H)),
    ],
    out_specs=[
        pl.BlockSpec((TILE_E, H), lambda i: (i, 0)),
        pl.BlockSpec((TILE_E, H), lambda i: (i, 0)),
    ],
    out_shape=[_bf16((E, H)), _f32((E, H))],
)


def _edge1_body(gdst_ref, gsrc_ref, eain_ref, weT_ref,
                eu2T_ref, eu2b_ref, f1T_ref, f1b_ref, f2T_ref, f2b_ref,
                eattr_ref, msg_ref):
    s = gsrc_ref[...]
    pd = _unpack_cat(gdst_ref[...])
    pj = _unpack_cat(s[:, :H])
    eterm = jnp.dot(eain_ref[...], weT_ref[...],
                    preferred_element_type=jnp.float32)
    ea = _ssp(eterm + pd + pj)
    eattr = _bdot(ea, eu2T_ref) + eu2b_ref[...]
    eattr_ref[...] = eattr.astype(jnp.bfloat16)
    w = _ssp(_bdot(eattr, f1T_ref) + f1b_ref[...])
    w = _ssp(_bdot(w, f2T_ref) + f2b_ref[...])
    msg_ref[...] = s[:, H:] * w


_edge1_call = pl.pallas_call(
    _edge1_body,
    grid=(E // TILE_E,),
    in_specs=[
        pl.BlockSpec((TILE_E, H), lambda i: (i, 0)),
        pl.BlockSpec((TILE_E, 2 * H), lambda i: (i, 0)),
        pl.BlockSpec((TILE_E, H), lambda i: (i, 0)),
        _full((H, 2 * H)),
        _full((2 * H, H)), _full((1, H)),
        _full((H, H)), _full((1, H)), _full((H, H)), _full((1, H)),
    ],
    out_specs=[
        pl.BlockSpec((TILE_E, H), lambda i: (i, 0)),
        pl.BlockSpec((TILE_E, H), lambda i: (i, 0)),
    ],
    out_shape=[_bf16((E, H)), _f32((E, H))],
)


def _edge2_body(gdst_ref, gsrc_ref, eain_ref, weT_ref,
                wcT_ref, bc_ref, f2T_ref, f2b_ref, msg_ref):
    s = gsrc_ref[...]
    pd = _unpack_cat(gdst_ref[...])
    pj = _unpack_cat(s[:, :H])
    eterm = jnp.dot(eain_ref[...], weT_ref[...],
                    preferred_element_type=jnp.float32)
    ea = _ssp(eterm + pd + pj)
    w = _ssp(_bdot(ea, wcT_ref) + bc_ref[...])
    w = _ssp(_bdot(w, f2T_ref) + f2b_ref[...])
    msg_ref[...] = s[:, H:] * w


_edge2_call = pl.pallas_call(
    _edge2_body,
    grid=(E // TILE_E,),
    in_specs=[
        pl.BlockSpec((TILE_E, H), lambda i: (i, 0)),
        pl.BlockSpec((TILE_E, 2 * H), lambda i: (i, 0)),
        pl.BlockSpec((TILE_E, H), lambda i: (i, 0)),
        _full((H, 2 * H)),
        _full((2 * H, H)), _full((1, H)),
        _full((H, H)), _full((1, H)),
    ],
    out_specs=pl.BlockSpec((TILE_E, H), lambda i: (i, 0)),
    out_shape=_f32((E, H)),
)


_MESH = plsc.VectorSubcoreMesh(core_axis_name="c", subcore_axis_name="s")


def _make_gather2(d1, d2):
    nch = EPW // CH
    assert nch % 2 == 1

    @functools.partial(
        pl.kernel,
        out_type=(_f32((E, d1)), _f32((E, d2))),
        mesh=_MESH,
        scratch_types=[
            pltpu.VMEM((CH,), jnp.int32), pltpu.VMEM((CH,), jnp.int32),
            pltpu.VMEM((CH,), jnp.int32), pltpu.VMEM((CH,), jnp.int32),
            pltpu.VMEM((CH, d1), jnp.float32),
            pltpu.VMEM((CH, d1), jnp.float32),
            pltpu.VMEM((CH, d2), jnp.float32),
            pltpu.VMEM((CH, d2), jnp.float32),
            pltpu.SemaphoreType.DMA, pltpu.SemaphoreType.DMA,
            pltpu.SemaphoreType.DMA, pltpu.SemaphoreType.DMA,
        ],
    )
    def gath(t1_hbm, i1_hbm, t2_hbm, i2_hbm, o1_hbm, o2_hbm,
             ia0, ia1, ib0, ib1, ra0, ra1, rb0, rb1, sa0, sa1, sb0, sb1):
        ia, ib = (ia0, ia1), (ib0, ib1)
        ra, rb = (ra0, ra1), (rb0, rb1)
        sa, sb = (sa0, sa1), (sb0, sb1)
        wid = lax.axis_index("s") * 2 + lax.axis_index("c")
        base = wid * EPW

        def load_start(buf, off):
            pltpu.sync_copy(i1_hbm.at[pl.ds(off, CH)], ia[buf])
            pltpu.sync_copy(i2_hbm.at[pl.ds(off, CH)], ib[buf])
            pltpu.async_copy(t1_hbm.at[ia[buf]], ra[buf], sa[buf])
            pltpu.async_copy(t2_hbm.at[ib[buf]], rb[buf], sb[buf])

        def wait_store(buf, off):
            pltpu.make_async_copy(t1_hbm.at[ia[buf]], ra[buf], sa[buf]).wait()
            pltpu.make_async_copy(t2_hbm.at[ib[buf]], rb[buf], sb[buf]).wait()
            pltpu.sync_copy(ra[buf], o1_hbm.at[pl.ds(off, CH)])
            pltpu.sync_copy(rb[buf], o2_hbm.at[pl.ds(off, CH)])

        load_start(0, base)

        @pl.loop(0, nch - 1, step=2)
        def _pair(ci):
            for b in range(2):
                off = base + (ci + b) * CH
                load_start(1 - b, off + CH)
                wait_store(b, off)

        wait_store(0, base + (nch - 1) * CH)

    return gath


_gather_tables = _make_gather2(H, 2 * H)
_gather_tables0 = _make_gather2(2 * H, 2 * H)


_NCH2 = EPW // CH2
assert _NCH2 % 2 == 1


@functools.partial(
    pl.kernel,
    out_type=_f32((2, NP, H)),
    mesh=_MESH,
    scratch_types=[
        pltpu.VMEM((CH2,), jnp.int32), pltpu.VMEM((CH2,), jnp.int32),
        pltpu.VMEM((CH2, H), jnp.float32), pltpu.VMEM((CH2, H), jnp.float32),
        pltpu.VMEM_SHARED((NP, H), jnp.float32),
        pltpu.SemaphoreType.DMA, pltpu.SemaphoreType.DMA,
    ],
)
def _scatter_call(msg_hbm, dst_hbm, zer_hbm, out_hbm,
                  i0, i1, r0, r1, acc, s0, s1):
    c = lax.axis_index("c")
    s = lax.axis_index("s")
    wid = s * 2 + c
    idx, rows, sem = (i0, i1), (r0, r1), (s0, s1)
    pltpu.sync_copy(zer_hbm, acc.at[pl.ds(s * RPT, RPT)])
    plsc.subcore_barrier()
    base = wid * EPW

    def load_start(buf, off):
        pltpu.sync_copy(dst_hbm.at[pl.ds(off, CH2)], idx[buf])
        pltpu.async_copy(msg_hbm.at[pl.ds(off, CH2)], rows[buf], sem[buf])

    def wait_add(buf, off):
        pltpu.make_async_copy(msg_hbm.at[pl.ds(off, CH2)], rows[buf],
                              sem[buf]).wait()
        pltpu.sync_copy(rows[buf], acc.at[idx[buf]], add=True)

    load_start(0, base)

    @pl.loop(0, _NCH2 - 1, step=2)
    def _pair(ci):
        for b in range(2):
            off = base + (ci + b) * CH2
            load_start(1 - b, off + CH2)
            wait_add(b, off)

    wait_add(0, base + (_NCH2 - 1) * CH2)
    plsc.subcore_barrier()
    pltpu.sync_copy(acc.at[pl.ds(s * RPT, RPT)],
                    out_hbm.at[c, pl.ds(s * RPT, RPT)])


def kernel(z, pos, edge_index, batch, params):
    src = edge_index[0].astype(jnp.int32)
    dst = edge_index[1].astype(jnp.int32)
    z2 = z.astype(jnp.int32).reshape(N, 1)
    batch2 = batch.astype(jnp.int32).reshape(N // TILE_N, 1, TILE_N)
    pospad = jnp.pad(pos.astype(jnp.float32), ((0, 0), (0, H - 3)))
    zer = jnp.zeros((RPT, H), jnp.float32)

    stop = CUTOFF - CUTOFF / NG
    offs = jnp.pad(jnp.linspace(0.0, stop, NG, dtype=jnp.float32),
                   (0, NGP - NG)).reshape(1, NGP)

    emb = jnp.pad(params['embedding'], ((0, NUM_EMB_P - NUM_EMB), (0, 0)))

    prep = []
    for t, p in enumerate(params['interactions']):
        w1 = p['eu1_W']
        wiT = w1[:, :H].T
        wjT = w1[:, H:2 * H].T
        weT = w1[:, 2 * H:].T
        if t == 0:
            weT = jnp.pad(weT, ((0, NGP - NG), (0, 0)))
        d = dict(
            wiT=wiT, b1=p['eu1_b'].reshape(1, 2 * H), wjT=wjT,
            cfT=p['cf_W'].T,
            weT=weT.astype(jnp.bfloat16),
            eu2T=p['eu2_W'].T.astype(jnp.bfloat16),
            eu2b=p['eu2_b'].reshape(1, H),
            f1T=p['f1_W'].T.astype(jnp.bfloat16),
            f1b=p['f1_b'].reshape(1, H),
            f2T=p['f2_W'].T.astype(jnp.bfloat16),
            f2b=p['f2_b'].reshape(1, H),
            sm1T=p['sm1_W'].T, sm1b=p['sm1_b'].reshape(1, H),
            sm2T=p['sm2_W'].T, sm2b=p['sm2_b'].reshape(1, H),
        )
        if t == NI - 1:
            d['wcT'] = (p['eu2_W'].T @ p['f1_W'].T).astype(jnp.bfloat16)
            d['bc'] = (p['eu2_b'] @ p['f1_W'].T
                       + p['f1_b']).reshape(1, H)
        prep.append(d)

    p0, p1, p2 = prep

    h0, pd, ps = _init_call(z2, pospad, emb, p0['wiT'], p0['b1'], p0['wjT'],
                            p0['cfT'])

    gd, gs = _gather_tables0(pd, dst, ps, src)
    eattr, msg_e = _edge0_call(gd, gs, offs, p0['weT'],
                               p0['eu2T'], p0['eu2b'], p0['f1T'], p0['f1b'],
                               p0['f2T'], p0['f2b'])
    msgp = _scatter_call(msg_e, dst, zer)
    h1, pd, ps = _node_call(msgp, h0, p0['sm1T'], p0['sm1b'], p0['sm2T'],
                            p0['sm2b'], p1['wiT'], p1['b1'], p1['wjT'],
                            p1['cfT'])

    gd, gs = _gather_tables(pd, dst, ps, src)
    eattr, msg_e = _edge1_call(gd, gs, eattr, p1['weT'],
                               p1['eu2T'], p1['eu2b'], p1['f1T'], p1['f1b'],
                               p1['f2T'], p1['f2b'])
    msgp = _scatter_call(msg_e, dst, zer)
    h2, pd, ps = _node_call(msgp, h1, p1['sm1T'], p1['sm1b'], p1['sm2T'],
                            p1['sm2b'], p2['wiT'], p2['b1'], p2['wjT'],
                            p2['cfT'])

    gd, gs = _gather_tables(pd, dst, ps, src)
    msg_e = _edge2_call(gd, gs, eattr, p2['weT'],
                        p2['wcT'], p2['bc'], p2['f2T'], p2['f2b'])
    msgp = _scatter_call(msg_e, dst, zer)

    lin1T = params['lin1_W'].T
    lin2T = params['lin2_W'].T
    out = _final_call(msgp, h2, p2['sm1T'], p2['sm1b'], p2['sm2T'],
                      p2['sm2b'], lin1T,
                      params['lin1_b'].reshape(1, H // 2), lin2T,
                      params['lin2_b'].reshape(1, 1), batch2)
    return out


# speedup vs baseline: 2.6533x; 1.0731x over previous
import functools

import jax
import jax.numpy as jnp
import numpy as np
from jax import lax
from jax.experimental import pallas as pl
from jax.experimental.pallas import tpu as pltpu
from jax.experimental.pallas import tpu_sc as plsc

N = 10000
E = 160000
H = 128
F = 128
NG = 150
NGP = 160
NI = 3
NUM_EMB = 100
NUM_EMB_P = 128
CUTOFF = 15.0
NG_GRAPHS = 512
LOG2 = float(np.log(2.0))

TILE_N = 1000
TILE_E = 2000

NW = 32
EPW = E // NW
CH = 40
CH2 = 40
NP = 10240
RPT = NP // 16


_LOG2E = 1.4426950408889634


def _ssp(x):
    t = jnp.exp2(jnp.abs(x) * (-_LOG2E))
    return jnp.maximum(x, 0.0) + jnp.log2(1.0 + t) * LOG2 - LOG2


def _f32(shape):
    return jax.ShapeDtypeStruct(shape, jnp.float32)


def _bf16(shape):
    return jax.ShapeDtypeStruct(shape, jnp.bfloat16)


_MASKHI = -65536


def _pack2(a, b):
    ab = lax.bitcast_convert_type(
        a.astype(jnp.bfloat16).astype(jnp.float32), jnp.int32)
    bb = lax.bitcast_convert_type(
        b.astype(jnp.bfloat16).astype(jnp.float32), jnp.int32)
    word = jnp.bitwise_or(lax.shift_right_logical(ab, 16),
                          jnp.bitwise_and(bb, _MASKHI))
    return lax.bitcast_convert_type(word, jnp.float32)


def _unpack2(p):
    u = lax.bitcast_convert_type(p, jnp.int32)
    a = lax.bitcast_convert_type(lax.shift_left(u, 16), jnp.float32)
    b = lax.bitcast_convert_type(jnp.bitwise_and(u, _MASKHI), jnp.float32)
    return a, b


def _unpack_cat(p):
    a, b = _unpack2(p)
    return jnp.concatenate([a, b], axis=1)


def _full(shape):
    return pl.BlockSpec(shape, lambda i: (0,) * len(shape))


def _dot(a, b):
    return jnp.dot(a, b, preferred_element_type=jnp.float32)


def _bdot(a, b_ref):
    return jnp.dot(a.astype(jnp.bfloat16), b_ref[...],
                   preferred_element_type=jnp.float32)


def _init_body(z_ref, pos_ref, emb_ref, wiT_ref, b1_ref, wjT_ref, cfT_ref,
               h_ref, pdst_ref, psrc_ref):
    zv = z_ref[...]
    oh = (zv == lax.broadcasted_iota(jnp.int32, (TILE_N, NUM_EMB_P), 1))
    h = _dot(oh.astype(jnp.float32), emb_ref[...])
    h_ref[...] = h
    pp = pos_ref[...]
    vi = _dot(h, wiT_ref[...]) + b1_ref[...]
    vj = _dot(h, wjT_ref[...])
    cf = _dot(h, cfT_ref[...])
    pdst_ref[...] = jnp.concatenate(
        [_pack2(vi[:, :H], vi[:, H:]), pp], axis=1)
    psrc_ref[...] = jnp.concatenate(
        [_pack2(vj[:, :H], vj[:, H:]),
         _pack2(cf[:, :H // 2], cf[:, H // 2:]),
         pp[:, :H // 2]], axis=1)


_init_call = pl.pallas_call(
    _init_body,
    grid=(N // TILE_N,),
    in_specs=[
        pl.BlockSpec((TILE_N, 1), lambda i: (i, 0)),
        pl.BlockSpec((TILE_N, H), lambda i: (i, 0)),
        _full((NUM_EMB_P, H)),
        _full((H, 2 * H)),
        _full((1, 2 * H)),
        _full((H, 2 * H)),
        _full((H, H)),
    ],
    out_specs=[
        pl.BlockSpec((TILE_N, H), lambda i: (i, 0)),
        pl.BlockSpec((TILE_N, 2 * H), lambda i: (i, 0)),
        pl.BlockSpec((TILE_N, 2 * H), lambda i: (i, 0)),
    ],
    out_shape=[_f32((N, H)), _f32((N, 2 * H)), _f32((N, 2 * H))],
)


def _node_body(msgp_ref, h_ref, sm1T_ref, sm1b_ref, sm2T_ref, sm2b_ref,
               wiT_ref, b1_ref, wjT_ref, cfT_ref,
               hout_ref, pdst_ref, psrc_ref):
    m = msgp_ref[...]
    msg = m[0] + m[1]
    u = _ssp(_dot(msg, sm1T_ref[...]) + sm1b_ref[...])
    h2 = h_ref[...] + _dot(u, sm2T_ref[...]) + sm2b_ref[...]
    hout_ref[...] = h2
    vi = _dot(h2, wiT_ref[...]) + b1_ref[...]
    vj = _dot(h2, wjT_ref[...])
    cf = _dot(h2, cfT_ref[...])
    pdst_ref[...] = _pack2(vi[:, :H], vi[:, H:])
    psrc_ref[...] = jnp.concatenate([_pack2(vj[:, :H], vj[:, H:]), cf],
                                    axis=1)


_node_call = pl.pallas_call(
    _node_body,
    grid=(N // TILE_N,),
    in_specs=[
        pl.BlockSpec((2, TILE_N, H), lambda i: (0, i, 0)),
        pl.BlockSpec((TILE_N, H), lambda i: (i, 0)),
        _full((H, H)), _full((1, H)), _full((H, H)), _full((1, H)),
        _full((H, 2 * H)), _full((1, 2 * H)), _full((H, 2 * H)),
        _full((H, H)),
    ],
    out_specs=[
        pl.BlockSpec((TILE_N, H), lambda i: (i, 0)),
        pl.BlockSpec((TILE_N, H), lambda i: (i, 0)),
        pl.BlockSpec((TILE_N, 2 * H), lambda i: (i, 0)),
    ],
    out_shape=[_f32((N, H)), _f32((N, H)), _f32((N, 2 * H))],
)


def _final_body(msgp_ref, h_ref, sm1T_ref, sm1b_ref, sm2T_ref, sm2b_ref,
                l1T_ref, l1b_ref, l2T_ref, l2b_ref, batch_ref, out_ref):
    i = pl.program_id(0)
    m = msgp_ref[...]
    msg = m[0] + m[1]
    u = _ssp(_dot(msg, sm1T_ref[...]) + sm1b_ref[...])
    h2 = h_ref[...] + _dot(u, sm2T_ref[...]) + sm2b_ref[...]
    no = _dot(_ssp(_dot(h2, l1T_ref[...]) + l1b_ref[...]), l2T_ref[...])
    no = no + l2b_ref[...]
    b2 = batch_ref[0]
    mask = (lax.broadcasted_iota(jnp.int32, (NG_GRAPHS, TILE_N), 0) == b2)
    part = _dot(mask.astype(jnp.float32), no)

    @pl.when(i == 0)
    def _zero():
        out_ref[...] = jnp.zeros_like(out_ref)

    out_ref[...] += part


_final_call = pl.pallas_call(
    _final_body,
    grid=(N // TILE_N,),
    in_specs=[
        pl.BlockSpec((2, TILE_N, H), lambda i: (0, i, 0)),
        pl.BlockSpec((TILE_N, H), lambda i: (i, 0)),
        _full((H, H)), _full((1, H)), _full((H, H)), _full((1, H)),
        _full((H, H // 2)), _full((1, H // 2)), _full((H // 2, 1)),
        _full((1, 1)),
        pl.BlockSpec((1, 1, TILE_N), lambda i: (i, 0, 0)),
    ],
    out_specs=pl.BlockSpec((NG_GRAPHS, 1), lambda i: (0, 0)),
    out_shape=_f32((NG_GRAPHS, 1)),
)


_COEFF = -0.5 / (CUTOFF / NG) ** 2


def _edge0_body(gdst_ref, gsrc_ref, offs_ref, weT_ref,
                eu2T_ref, eu2b_ref, f1T_ref, f1b_ref, f2T_ref, f2b_ref,
                eattr_ref, msg_ref):
    gd = gdst_ref[...]
    s = gsrc_ref[...]
    d = gd[:, H:H + H // 2] - s[:, H + H // 2:]
    dist = jnp.sqrt(jnp.sum(d * d, axis=1, keepdims=True))
    dif = dist - offs_ref[...]
    g = jnp.exp2(dif * dif * (_COEFF * _LOG2E))
    pd = _unpack_cat(gd[:, :H])
    pj = _unpack_cat(s[:, :H])
    cf = _unpack_cat(s[:, H:H + H // 2])
    ea = _ssp(_bdot(g, weT_ref) + pd + pj)
    eattr = _bdot(ea, eu2T_ref) + eu2b_ref[...]
    eattr_ref[...] = eattr.astype(jnp.bfloat16)
    w = _ssp(_bdot(eattr, f1T_ref) + f1b_ref[...])
    w = _ssp(_bdot(w, f2T_ref) + f2b_ref[...])
    msg_ref[...] = cf * w


_edge0_call = pl.pallas_call(
    _edge0_body,
    grid=(E // TILE_E,),
    in_specs=[
        pl.BlockSpec((TILE_E, 2 * H), lambda i: (i, 0)),
        pl.BlockSpec((TILE_E, 2 * H), lambda i: (i, 0)),
        _full((1, NGP)), _full((NGP, 2 * H)),
        _full((2 * H, H)), _full((1, H)),
        _full((H, H)), _full((1, H)), _full((H, H)), _full((1, H)),
    ],
    out_specs=[
        pl.BlockSpec((TILE_E, H), lambda i: (i, 0)),
        pl.BlockSpec((TILE_E, H), lambda i: (i, 0)),
    ],
    out_shape=[_bf16((E, H)), _f32((E, H))],
)


def _edge1_body(gdst_ref, gsrc_ref, eain_ref, weT_ref,
                eu2T_ref, eu2b_ref, f1T_ref, f1b_ref, f2T_ref, f2b_ref,
                eattr_ref, msg_ref):
    s = gsrc_ref[...]
    pd = _unpack_cat(gdst_ref[...])
    pj = _unpack_cat(s[:, :H])
    eterm = jnp.dot(eain_ref[...], weT_ref[...],
                    preferred_element_type=jnp.float32)
    ea = _ssp(eterm + pd + pj)
    eattr = _bdot(ea, eu2T_ref) + eu2b_ref[...]
    eattr_ref[...] = eattr.astype(jnp.bfloat16)
    w = _ssp(_bdot(eattr, f1T_ref) + f1b_ref[...])
    w = _ssp(_bdot(w, f2T_ref) + f2b_ref[...])
    msg_ref[...] = s[:, H:] * w


_edge1_call = pl.pallas_call(
    _edge1_body,
    grid=(E // TILE_E,),
    in_specs=[
        pl.BlockSpec((TILE_E, H), lambda i: (i, 0)),
        pl.BlockSpec((TILE_E, 2 * H), lambda i: (i, 0)),
        pl.BlockSpec((TILE_E, H), lambda i: (i, 0)),
        _full((H, 2 * H)),
        _full((2 * H, H)), _full((1, H)),
        _full((H, H)), _full((1, H)), _full((H, H)), _full((1, H)),
    ],
    out_specs=[
        pl.BlockSpec((TILE_E, H), lambda i: (i, 0)),
        pl.BlockSpec((TILE_E, H), lambda i: (i, 0)),
    ],
    out_shape=[_bf16((E, H)), _f32((E, H))],
)


def _edge2_body(gdst_ref, gsrc_ref, eain_ref, weT_ref,
                wcT_ref, bc_ref, f2T_ref, f2b_ref, msg_ref):
    s = gsrc_ref[...]
    pd = _unpack_cat(gdst_ref[...])
    pj = _unpack_cat(s[:, :H])
    eterm = jnp.dot(eain_ref[...], weT_ref[...],
                    preferred_element_type=jnp.float32)
    ea = _ssp(eterm + pd + pj)
    w = _ssp(_bdot(ea, wcT_ref) + bc_ref[...])
    w = _ssp(_bdot(w, f2T_ref) + f2b_ref[...])
    msg_ref[...] = s[:, H:] * w


_edge2_call = pl.pallas_call(
    _edge2_body,
    grid=(E // TILE_E,),
    in_specs=[
        pl.BlockSpec((TILE_E, H), lambda i: (i, 0)),
        pl.BlockSpec((TILE_E, 2 * H), lambda i: (i, 0)),
        pl.BlockSpec((TILE_E, H), lambda i: (i, 0)),
        _full((H, 2 * H)),
        _full((2 * H, H)), _full((1, H)),
        _full((H, H)), _full((1, H)),
    ],
    out_specs=pl.BlockSpec((TILE_E, H), lambda i: (i, 0)),
    out_shape=_f32((E, H)),
)


_MESH = plsc.VectorSubcoreMesh(core_axis_name="c", subcore_axis_name="s")


def _make_gather2(d1, d2):
    nch = EPW // CH
    assert nch % 2 == 1

    @functools.partial(
        pl.kernel,
        out_type=(_f32((E, d1)), _f32((E, d2))),
        mesh=_MESH,
        scratch_types=[
            pltpu.VMEM((CH,), jnp.int32), pltpu.VMEM((CH,), jnp.int32),
            pltpu.VMEM((CH,), jnp.int32), pltpu.VMEM((CH,), jnp.int32),
            pltpu.VMEM((CH, d1), jnp.float32),
            pltpu.VMEM((CH, d1), jnp.float32),
            pltpu.VMEM((CH, d2), jnp.float32),
            pltpu.VMEM((CH, d2), jnp.float32),
            pltpu.SemaphoreType.DMA, pltpu.SemaphoreType.DMA,
            pltpu.SemaphoreType.DMA, pltpu.SemaphoreType.DMA,
        ],
    )
    def gath(t1_hbm, i1_hbm, t2_hbm, i2_hbm, o1_hbm, o2_hbm,
             ia0, ia1, ib0, ib1, ra0, ra1, rb0, rb1, sa0, sa1, sb0, sb1):
        ia, ib = (ia0, ia1), (ib0, ib1)
        ra, rb = (ra0, ra1), (rb0, rb1)
        sa, sb = (sa0, sa1), (sb0, sb1)
        wid = lax.axis_index("s") * 2 + lax.axis_index("c")
        base = wid * EPW

        def load_start(buf, off):
            pltpu.sync_copy(i1_hbm.at[pl.ds(off, CH)], ia[buf])
            pltpu.sync_copy(i2_hbm.at[pl.ds(off, CH)], ib[buf])
            pltpu.async_copy(t1_hbm.at[ia[buf]], ra[buf], sa[buf])
            pltpu.async_copy(t2_hbm.at[ib[buf]], rb[buf], sb[buf])

        def wait_store(buf, off):
            pltpu.make_async_copy(t1_hbm.at[ia[buf]], ra[buf], sa[buf]).wait()
            pltpu.make_async_copy(t2_hbm.at[ib[buf]], rb[buf], sb[buf]).wait()
            pltpu.sync_copy(ra[buf], o1_hbm.at[pl.ds(off, CH)])
            pltpu.sync_copy(rb[buf], o2_hbm.at[pl.ds(off, CH)])

        load_start(0, base)

        @pl.loop(0, nch - 1, step=2)
        def _pair(ci):
            for b in range(2):
                off = base + (ci + b) * CH
                load_start(1 - b, off + CH)
                wait_store(b, off)

        wait_store(0, base + (nch - 1) * CH)

    return gath


_gather_tables = _make_gather2(H, 2 * H)
_gather_tables0 = _make_gather2(2 * H, 2 * H)


_NCH2 = EPW // CH2
assert _NCH2 % 2 == 1


@functools.partial(
    pl.kernel,
    out_type=_f32((2, NP, H)),
    mesh=_MESH,
    scratch_types=[
        pltpu.VMEM((CH2,), jnp.int32), pltpu.VMEM((CH2,), jnp.int32),
        pltpu.VMEM((CH2, H), jnp.float32), pltpu.VMEM((CH2, H), jnp.float32),
        pltpu.VMEM_SHARED((NP, H), jnp.float32),
        pltpu.SemaphoreType.DMA, pltpu.SemaphoreType.DMA,
    ],
)
def _scatter_call(msg_hbm, dst_hbm, zer_hbm, out_hbm,
                  i0, i1, r0, r1, acc, s0, s1):
    c = lax.axis_index("c")
    s = lax.axis_index("s")
    wid = s * 2 + c
    idx, rows, sem = (i0, i1), (r0, r1), (s0, s1)
    pltpu.sync_copy(zer_hbm, acc.at[pl.ds(s * RPT, RPT)])
    plsc.subcore_barrier()
    base = wid * EPW

    def load_start(buf, off):
        pltpu.sync_copy(dst_hbm.at[pl.ds(off, CH2)], idx[buf])
        pltpu.async_copy(msg_hbm.at[pl.ds(off, CH2)], rows[buf], sem[buf])

    def wait_add(buf, off):
        pltpu.make_async_copy(msg_hbm.at[pl.ds(off, CH2)], rows[buf],
                              sem[buf]).wait()
        pltpu.sync_copy(rows[buf], acc.at[idx[buf]], add=True)

    load_start(0, base)

    @pl.loop(0, _NCH2 - 1, step=2)
    def _pair(ci):
        for b in range(2):
            off = base + (ci + b) * CH2
            load_start(1 - b, off + CH2)
            wait_add(b, off)

    wait_add(0, base + (_NCH2 - 1) * CH2)
    plsc.subcore_barrier()
    pltpu.sync_copy(acc.at[pl.ds(s * RPT, RPT)],
                    out_hbm.at[c, pl.ds(s * RPT, RPT)])


def kernel(z, pos, edge_index, batch, params):
    src = edge_index[0].astype(jnp.int32)
    dst = edge_index[1].astype(jnp.int32)
    z2 = z.astype(jnp.int32).reshape(N, 1)
    batch2 = batch.astype(jnp.int32).reshape(N // TILE_N, 1, TILE_N)
    pospad = jnp.pad(pos.astype(jnp.float32), ((0, 0), (0, H - 3)))
    zer = jnp.zeros((RPT, H), jnp.float32)

    stop = CUTOFF - CUTOFF / NG
    offs = jnp.pad(jnp.linspace(0.0, stop, NG, dtype=jnp.float32),
                   (0, NGP - NG)).reshape(1, NGP)

    emb = jnp.pad(params['embedding'], ((0, NUM_EMB_P - NUM_EMB), (0, 0)))

    prep = []
    for t, p in enumerate(params['interactions']):
        w1 = p['eu1_W']
        wiT = w1[:, :H].T
        wjT = w1[:, H:2 * H].T
        weT = w1[:, 2 * H:].T
        if t == 0:
            weT = jnp.pad(weT, ((0, NGP - NG), (0, 0)))
        d = dict(
            wiT=wiT, b1=p['eu1_b'].reshape(1, 2 * H), wjT=wjT,
            cfT=p['cf_W'].T,
            weT=weT.astype(jnp.bfloat16),
            eu2T=p['eu2_W'].T.astype(jnp.bfloat16),
            eu2b=p['eu2_b'].reshape(1, H),
            f1T=p['f1_W'].T.astype(jnp.bfloat16),
            f1b=p['f1_b'].reshape(1, H),
            f2T=p['f2_W'].T.astype(jnp.bfloat16),
            f2b=p['f2_b'].reshape(1, H),
            sm1T=p['sm1_W'].T, sm1b=p['sm1_b'].reshape(1, H),
            sm2T=p['sm2_W'].T, sm2b=p['sm2_b'].reshape(1, H),
        )
        if t == NI - 1:
            d['wcT'] = (p['eu2_W'].T @ p['f1_W'].T).astype(jnp.bfloat16)
            d['bc'] = (p['eu2_b'] @ p['f1_W'].T
                       + p['f1_b']).reshape(1, H)
        prep.append(d)

    p0, p1, p2 = prep

    h0, pd, ps = _init_call(z2, pospad, emb, p0['wiT'], p0['b1'], p0['wjT'],
                            p0['cfT'])

    gd, gs = _gather_tables0(pd, dst, ps, src)
    eattr, msg_e = _edge0_call(gd, gs, offs, p0['weT'],
                               p0['eu2T'], p0['eu2b'], p0['f1T'], p0['f1b'],
                               p0['f2T'], p0['f2b'])
    msgp = _scatter_call(msg_e, dst, zer)
    h1, pd, ps = _node_call(msgp, h0, p0['sm1T'], p0['sm1b'], p0['sm2T'],
                            p0['sm2b'], p1['wiT'], p1['b1'], p1['wjT'],
                            p1['cfT'])

    gd, gs = _gather_tables(pd, dst, ps, src)
    eattr, msg_e = _edge1_call(gd, gs, eattr, p1['weT'],
                               p1['eu2T'], p1['eu2b'], p1['f1T'], p1['f1b'],
                               p1['f2T'], p1['f2b'])
    msgp = _scatter_call(msg_e, dst, zer)
    h2, pd, ps = _node_call(msgp, h1, p1['sm1T'], p1['sm1b'], p1['sm2T'],
                            p1['sm2b'], p2['wiT'], p2['b1'], p2['wjT'],
                            p2['cfT'])

    gd, gs = _gather_tables(pd, dst, ps, src)
    msg_e = _edge2_call(gd, gs, eattr, p2['weT'],
                        p2['wcT'], p2['bc'], p2['f2T'], p2['f2b'])
    msgp = _scatter_call(msg_e, dst, zer)

    lin1T = params['lin1_W'].T
    lin2T = params['lin2_W'].T
    out = _final_call(msgp, h2, p2['sm1T'], p2['sm1b'], p2['sm2T'],
                      p2['sm2b'], lin1T,
                      params['lin1_b'].reshape(1, H // 2), lin2T,
                      params['lin2_b'].reshape(1, 1), batch2)
    return out


# speedup vs baseline: 2.7384x; 1.0321x over previous
import functools

import jax
import jax.numpy as jnp
import numpy as np
from jax import lax
from jax.experimental import pallas as pl
from jax.experimental.pallas import tpu as pltpu
from jax.experimental.pallas import tpu_sc as plsc

N = 10000
E = 160000
H = 128
F = 128
NG = 150
NGP = 160
NI = 3
NUM_EMB = 100
NUM_EMB_P = 128
CUTOFF = 15.0
NG_GRAPHS = 512
LOG2 = float(np.log(2.0))

TILE_N = 2000
TILE_E = 4000

NW = 32
EPW = E // NW
CH = 40
CH2 = 40
NP = 10240
RPT = NP // 16


_LOG2E = 1.4426950408889634


def _ssp(x):
    t = jnp.exp2(jnp.abs(x) * (-_LOG2E))
    return jnp.maximum(x, 0.0) + jnp.log2(1.0 + t) * LOG2 - LOG2


_sspb = _ssp


def _f32(shape):
    return jax.ShapeDtypeStruct(shape, jnp.float32)


def _bf16(shape):
    return jax.ShapeDtypeStruct(shape, jnp.bfloat16)


_MASKHI = -65536


def _pack2(a, b):
    ab = lax.bitcast_convert_type(
        a.astype(jnp.bfloat16).astype(jnp.float32), jnp.int32)
    bb = lax.bitcast_convert_type(
        b.astype(jnp.bfloat16).astype(jnp.float32), jnp.int32)
    word = jnp.bitwise_or(lax.shift_right_logical(ab, 16),
                          jnp.bitwise_and(bb, _MASKHI))
    return lax.bitcast_convert_type(word, jnp.float32)


def _unpack2(p):
    u = lax.bitcast_convert_type(p, jnp.int32)
    a = lax.bitcast_convert_type(lax.shift_left(u, 16), jnp.float32)
    b = lax.bitcast_convert_type(jnp.bitwise_and(u, _MASKHI), jnp.float32)
    return a, b


def _unpack_cat(p):
    a, b = _unpack2(p)
    return jnp.concatenate([a, b], axis=1)


def _full(shape):
    return pl.BlockSpec(shape, lambda i: (0,) * len(shape))


def _dot(a, b):
    return jnp.dot(a, b, preferred_element_type=jnp.float32)


def _bdot(a, b_ref):
    return jnp.dot(a.astype(jnp.bfloat16), b_ref[...],
                   preferred_element_type=jnp.float32)


def _init_body(z_ref, pos_ref, emb_ref, wiT_ref, b1_ref, wjT_ref, cfT_ref,
               h_ref, pdst_ref, psrc_ref):
    zv = z_ref[...]
    oh = (zv == lax.broadcasted_iota(jnp.int32, (TILE_N, NUM_EMB_P), 1))
    h = _dot(oh.astype(jnp.float32), emb_ref[...])
    h_ref[...] = h
    pp = pos_ref[...]
    vi = _dot(h, wiT_ref[...]) + b1_ref[...]
    vj = _dot(h, wjT_ref[...])
    cf = _dot(h, cfT_ref[...])
    pdst_ref[...] = jnp.concatenate(
        [_pack2(vi[:, :H], vi[:, H:]), pp], axis=1)
    psrc_ref[...] = jnp.concatenate(
        [_pack2(vj[:, :H], vj[:, H:]),
         _pack2(cf[:, :H // 2], cf[:, H // 2:]),
         pp[:, :H // 2]], axis=1)


_init_call = pl.pallas_call(
    _init_body,
    grid=(N // TILE_N,),
    in_specs=[
        pl.BlockSpec((TILE_N, 1), lambda i: (i, 0)),
        pl.BlockSpec((TILE_N, H), lambda i: (i, 0)),
        _full((NUM_EMB_P, H)),
        _full((H, 2 * H)),
        _full((1, 2 * H)),
        _full((H, 2 * H)),
        _full((H, H)),
    ],
    out_specs=[
        pl.BlockSpec((TILE_N, H), lambda i: (i, 0)),
        pl.BlockSpec((TILE_N, 2 * H), lambda i: (i, 0)),
        pl.BlockSpec((TILE_N, 2 * H), lambda i: (i, 0)),
    ],
    out_shape=[_f32((N, H)), _f32((N, 2 * H)), _f32((N, 2 * H))],
)


def _node_body(msgp_ref, h_ref, sm1T_ref, sm1b_ref, sm2T_ref, sm2b_ref,
               wiT_ref, b1_ref, wjT_ref, cfT_ref,
               hout_ref, pdst_ref, psrc_ref):
    m = msgp_ref[...]
    msg = m[0] + m[1]
    u = _ssp(_dot(msg, sm1T_ref[...]) + sm1b_ref[...])
    h2 = h_ref[...] + _dot(u, sm2T_ref[...]) + sm2b_ref[...]
    hout_ref[...] = h2
    vi = _dot(h2, wiT_ref[...]) + b1_ref[...]
    vj = _dot(h2, wjT_ref[...])
    cf = _dot(h2, cfT_ref[...])
    pdst_ref[...] = _pack2(vi[:, :H], vi[:, H:])
    psrc_ref[...] = jnp.concatenate([_pack2(vj[:, :H], vj[:, H:]), cf],
                                    axis=1)


_node_call = pl.pallas_call(
    _node_body,
    grid=(N // TILE_N,),
    in_specs=[
        pl.BlockSpec((2, TILE_N, H), lambda i: (0, i, 0)),
        pl.BlockSpec((TILE_N, H), lambda i: (i, 0)),
        _full((H, H)), _full((1, H)), _full((H, H)), _full((1, H)),
        _full((H, 2 * H)), _full((1, 2 * H)), _full((H, 2 * H)),
        _full((H, H)),
    ],
    out_specs=[
        pl.BlockSpec((TILE_N, H), lambda i: (i, 0)),
        pl.BlockSpec((TILE_N, H), lambda i: (i, 0)),
        pl.BlockSpec((TILE_N, 2 * H), lambda i: (i, 0)),
    ],
    out_shape=[_f32((N, H)), _f32((N, H)), _f32((N, 2 * H))],
)


def _final_body(msgp_ref, h_ref, sm1T_ref, sm1b_ref, sm2T_ref, sm2b_ref,
                l1T_ref, l1b_ref, l2T_ref, l2b_ref, batch_ref, out_ref):
    i = pl.program_id(0)
    m = msgp_ref[...]
    msg = m[0] + m[1]
    u = _ssp(_dot(msg, sm1T_ref[...]) + sm1b_ref[...])
    h2 = h_ref[...] + _dot(u, sm2T_ref[...]) + sm2b_ref[...]
    no = _dot(_ssp(_dot(h2, l1T_ref[...]) + l1b_ref[...]), l2T_ref[...])
    no = no + l2b_ref[...]
    b2 = batch_ref[0]
    mask = (lax.broadcasted_iota(jnp.int32, (NG_GRAPHS, TILE_N), 0) == b2)
    part = _dot(mask.astype(jnp.float32), no)

    @pl.when(i == 0)
    def _zero():
        out_ref[...] = jnp.zeros_like(out_ref)

    out_ref[...] += part


_final_call = pl.pallas_call(
    _final_body,
    grid=(N // TILE_N,),
    in_specs=[
        pl.BlockSpec((2, TILE_N, H), lambda i: (0, i, 0)),
        pl.BlockSpec((TILE_N, H), lambda i: (i, 0)),
        _full((H, H)), _full((1, H)), _full((H, H)), _full((1, H)),
        _full((H, H // 2)), _full((1, H // 2)), _full((H // 2, 1)),
        _full((1, 1)),
        pl.BlockSpec((1, 1, TILE_N), lambda i: (i, 0, 0)),
    ],
    out_specs=pl.BlockSpec((NG_GRAPHS, 1), lambda i: (0, 0)),
    out_shape=_f32((NG_GRAPHS, 1)),
)


_COEFF = -0.5 / (CUTOFF / NG) ** 2


def _edge0_body(gdst_ref, gsrc_ref, offs_ref, weT_ref,
                eu2T_ref, eu2b_ref, f1T_ref, f1b_ref, f2T_ref, f2b_ref,
                eattr_ref, msg_ref):
    gd = gdst_ref[...]
    s = gsrc_ref[...]
    d = gd[:, H:H + H // 2] - s[:, H + H // 2:]
    dist = jnp.sqrt(jnp.sum(d * d, axis=1, keepdims=True))
    dif = dist - offs_ref[...]
    g = jnp.exp2((dif * dif * (_COEFF * _LOG2E)).astype(jnp.bfloat16))
    pd = _unpack_cat(gd[:, :H])
    pj = _unpack_cat(s[:, :H])
    cf = _unpack_cat(s[:, H:H + H // 2])
    ea = _sspb(_bdot(g, weT_ref) + pd + pj)
    eattr = _bdot(ea, eu2T_ref) + eu2b_ref[...]
    eattr_ref[...] = eattr.astype(jnp.bfloat16)
    w = _sspb(_bdot(eattr, f1T_ref) + f1b_ref[...])
    w = _sspb(_bdot(w, f2T_ref) + f2b_ref[...])
    msg_ref[...] = cf * w.astype(jnp.float32)


_edge0_call = pl.pallas_call(
    _edge0_body,
    grid=(E // TILE_E,),
    in_specs=[
        pl.BlockSpec((TILE_E, 2 * H), lambda i: (i, 0)),
        pl.BlockSpec((TILE_E, 2 * H), lambda i: (i, 0)),
        _full((1, NGP)), _full((NGP, 2 * H)),
        _full((2 * H, H)), _full((1, H)),
        _full((H, H)), _full((1, H)), _full((H, H)), _full((1, H)),
    ],
    out_specs=[
        pl.BlockSpec((TILE_E, H), lambda i: (i, 0)),
        pl.BlockSpec((TILE_E, H), lambda i: (i, 0)),
    ],
    out_shape=[_bf16((E, H)), _f32((E, H))],
)


def _edge1_body(gdst_ref, gsrc_ref, eain_ref, weT_ref,
                eu2T_ref, eu2b_ref, f1T_ref, f1b_ref, f2T_ref, f2b_ref,
                eattr_ref, msg_ref):
    s = gsrc_ref[...]
    pd = _unpack_cat(gdst_ref[...])
    pj = _unpack_cat(s[:, :H])
    eterm = jnp.dot(eain_ref[...], weT_ref[...],
                    preferred_element_type=jnp.float32)
    ea = _sspb(eterm + pd + pj)
    eattr = _bdot(ea, eu2T_ref) + eu2b_ref[...]
    eattr_ref[...] = eattr.astype(jnp.bfloat16)
    w = _sspb(_bdot(eattr, f1T_ref) + f1b_ref[...])
    w = _sspb(_bdot(w, f2T_ref) + f2b_ref[...])
    msg_ref[...] = s[:, H:] * w.astype(jnp.float32)


_edge1_call = pl.pallas_call(
    _edge1_body,
    grid=(E // TILE_E,),
    in_specs=[
        pl.BlockSpec((TILE_E, H), lambda i: (i, 0)),
        pl.BlockSpec((TILE_E, 2 * H), lambda i: (i, 0)),
        pl.BlockSpec((TILE_E, H), lambda i: (i, 0)),
        _full((H, 2 * H)),
        _full((2 * H, H)), _full((1, H)),
        _full((H, H)), _full((1, H)), _full((H, H)), _full((1, H)),
    ],
    out_specs=[
        pl.BlockSpec((TILE_E, H), lambda i: (i, 0)),
        pl.BlockSpec((TILE_E, H), lambda i: (i, 0)),
    ],
    out_shape=[_bf16((E, H)), _f32((E, H))],
)


def _edge2_body(gdst_ref, gsrc_ref, eain_ref, weT_ref,
                wcT_ref, bc_ref, f2T_ref, f2b_ref, msg_ref):
    s = gsrc_ref[...]
    pd = _unpack_cat(gdst_ref[...])
    pj = _unpack_cat(s[:, :H])
    eterm = jnp.dot(eain_ref[...], weT_ref[...],
                    preferred_element_type=jnp.float32)
    ea = _sspb(eterm + pd + pj)
    w = _sspb(_bdot(ea, wcT_ref) + bc_ref[...])
    w = _sspb(_bdot(w, f2T_ref) + f2b_ref[...])
    msg_ref[...] = s[:, H:] * w.astype(jnp.float32)


_edge2_call = pl.pallas_call(
    _edge2_body,
    grid=(E // TILE_E,),
    in_specs=[
        pl.BlockSpec((TILE_E, H), lambda i: (i, 0)),
        pl.BlockSpec((TILE_E, 2 * H), lambda i: (i, 0)),
        pl.BlockSpec((TILE_E, H), lambda i: (i, 0)),
        _full((H, 2 * H)),
        _full((2 * H, H)), _full((1, H)),
        _full((H, H)), _full((1, H)),
    ],
    out_specs=pl.BlockSpec((TILE_E, H), lambda i: (i, 0)),
    out_shape=_f32((E, H)),
)


_MESH = plsc.VectorSubcoreMesh(core_axis_name="c", subcore_axis_name="s")


def _make_gather2(d1, d2):
    nch = EPW // CH
    assert nch % 2 == 1

    @functools.partial(
        pl.kernel,
        out_type=(_f32((E, d1)), _f32((E, d2))),
        mesh=_MESH,
        scratch_types=[
            pltpu.VMEM((CH,), jnp.int32), pltpu.VMEM((CH,), jnp.int32),
            pltpu.VMEM((CH,), jnp.int32), pltpu.VMEM((CH,), jnp.int32),
            pltpu.VMEM((CH, d1), jnp.float32),
            pltpu.VMEM((CH, d1), jnp.float32),
            pltpu.VMEM((CH, d2), jnp.float32),
            pltpu.VMEM((CH, d2), jnp.float32),
            pltpu.SemaphoreType.DMA, pltpu.SemaphoreType.DMA,
            pltpu.SemaphoreType.DMA, pltpu.SemaphoreType.DMA,
        ],
    )
    def gath(t1_hbm, i1_hbm, t2_hbm, i2_hbm, o1_hbm, o2_hbm,
             ia0, ia1, ib0, ib1, ra0, ra1, rb0, rb1, sa0, sa1, sb0, sb1):
        ia, ib = (ia0, ia1), (ib0, ib1)
        ra, rb = (ra0, ra1), (rb0, rb1)
        sa, sb = (sa0, sa1), (sb0, sb1)
        wid = lax.axis_index("s") * 2 + lax.axis_index("c")
        base = wid * EPW

        def load_start(buf, off):
            pltpu.sync_copy(i1_hbm.at[pl.ds(off, CH)], ia[buf])
            pltpu.sync_copy(i2_hbm.at[pl.ds(off, CH)], ib[buf])
            pltpu.async_copy(t1_hbm.at[ia[buf]], ra[buf], sa[buf])
            pltpu.async_copy(t2_hbm.at[ib[buf]], rb[buf], sb[buf])

        def wait_store(buf, off):
            pltpu.make_async_copy(t1_hbm.at[ia[buf]], ra[buf], sa[buf]).wait()
            pltpu.make_async_copy(t2_hbm.at[ib[buf]], rb[buf], sb[buf]).wait()
            pltpu.sync_copy(ra[buf], o1_hbm.at[pl.ds(off, CH)])
            pltpu.sync_copy(rb[buf], o2_hbm.at[pl.ds(off, CH)])

        load_start(0, base)

        @pl.loop(0, nch - 1, step=2)
        def _pair(ci):
            for b in range(2):
                off = base + (ci + b) * CH
                load_start(1 - b, off + CH)
                wait_store(b, off)

        wait_store(0, base + (nch - 1) * CH)

    return gath


_gather_tables = _make_gather2(H, 2 * H)
_gather_tables0 = _make_gather2(2 * H, 2 * H)


_NCH2 = EPW // CH2
assert _NCH2 % 2 == 1


@functools.partial(
    pl.kernel,
    out_type=_f32((2, NP, H)),
    mesh=_MESH,
    scratch_types=[
        pltpu.VMEM((CH2,), jnp.int32), pltpu.VMEM((CH2,), jnp.int32),
        pltpu.VMEM((CH2, H), jnp.float32), pltpu.VMEM((CH2, H), jnp.float32),
        pltpu.VMEM_SHARED((NP, H), jnp.float32),
        pltpu.SemaphoreType.DMA, pltpu.SemaphoreType.DMA,
    ],
)
def _scatter_call(msg_hbm, dst_hbm, zer_hbm, out_hbm,
                  i0, i1, r0, r1, acc, s0, s1):
    c = lax.axis_index("c")
    s = lax.axis_index("s")
    wid = s * 2 + c
    idx, rows, sem = (i0, i1), (r0, r1), (s0, s1)
    pltpu.sync_copy(zer_hbm, acc.at[pl.ds(s * RPT, RPT)])
    plsc.subcore_barrier()
    base = wid * EPW

    def load_start(buf, off):
        pltpu.sync_copy(dst_hbm.at[pl.ds(off, CH2)], idx[buf])
        pltpu.async_copy(msg_hbm.at[pl.ds(off, CH2)], rows[buf], sem[buf])

    def wait_add(buf, off):
        pltpu.make_async_copy(msg_hbm.at[pl.ds(off, CH2)], rows[buf],
                              sem[buf]).wait()
        pltpu.sync_copy(rows[buf], acc.at[idx[buf]], add=True)

    load_start(0, base)

    @pl.loop(0, _NCH2 - 1, step=2)
    def _pair(ci):
        for b in range(2):
            off = base + (ci + b) * CH2
            load_start(1 - b, off + CH2)
            wait_add(b, off)

    wait_add(0, base + (_NCH2 - 1) * CH2)
    plsc.subcore_barrier()
    pltpu.sync_copy(acc.at[pl.ds(s * RPT, RPT)],
                    out_hbm.at[c, pl.ds(s * RPT, RPT)])


def kernel(z, pos, edge_index, batch, params):
    src = edge_index[0].astype(jnp.int32)
    dst = edge_index[1].astype(jnp.int32)
    z2 = z.astype(jnp.int32).reshape(N, 1)
    batch2 = batch.astype(jnp.int32).reshape(N // TILE_N, 1, TILE_N)
    pospad = jnp.pad(pos.astype(jnp.float32), ((0, 0), (0, H - 3)))
    zer = jnp.zeros((RPT, H), jnp.float32)

    stop = CUTOFF - CUTOFF / NG
    offs = jnp.pad(jnp.linspace(0.0, stop, NG, dtype=jnp.float32),
                   (0, NGP - NG)).reshape(1, NGP)

    emb = jnp.pad(params['embedding'], ((0, NUM_EMB_P - NUM_EMB), (0, 0)))

    prep = []
    for t, p in enumerate(params['interactions']):
        w1 = p['eu1_W']
        wiT = w1[:, :H].T
        wjT = w1[:, H:2 * H].T
        weT = w1[:, 2 * H:].T
        if t == 0:
            weT = jnp.pad(weT, ((0, NGP - NG), (0, 0)))
        d = dict(
            wiT=wiT, b1=p['eu1_b'].reshape(1, 2 * H), wjT=wjT,
            cfT=p['cf_W'].T,
            weT=weT.astype(jnp.bfloat16),
            eu2T=p['eu2_W'].T.astype(jnp.bfloat16),
            eu2b=p['eu2_b'].reshape(1, H),
            f1T=p['f1_W'].T.astype(jnp.bfloat16),
            f1b=p['f1_b'].reshape(1, H),
            f2T=p['f2_W'].T.astype(jnp.bfloat16),
            f2b=p['f2_b'].reshape(1, H),
            sm1T=p['sm1_W'].T, sm1b=p['sm1_b'].reshape(1, H),
            sm2T=p['sm2_W'].T, sm2b=p['sm2_b'].reshape(1, H),
        )
        if t == NI - 1:
            d['wcT'] = (p['eu2_W'].T @ p['f1_W'].T).astype(jnp.bfloat16)
            d['bc'] = (p['eu2_b'] @ p['f1_W'].T
                       + p['f1_b']).reshape(1, H)
        prep.append(d)

    p0, p1, p2 = prep

    h0, pd, ps = _init_call(z2, pospad, emb, p0['wiT'], p0['b1'], p0['wjT'],
                            p0['cfT'])

    gd, gs = _gather_tables0(pd, dst, ps, src)
    eattr, msg_e = _edge0_call(gd, gs, offs, p0['weT'],
                               p0['eu2T'], p0['eu2b'], p0['f1T'], p0['f1b'],
                               p0['f2T'], p0['f2b'])
    msgp = _scatter_call(msg_e, dst, zer)
    h1, pd, ps = _node_call(msgp, h0, p0['sm1T'], p0['sm1b'], p0['sm2T'],
                            p0['sm2b'], p1['wiT'], p1['b1'], p1['wjT'],
                            p1['cfT'])

    gd, gs = _gather_tables(pd, dst, ps, src)
    eattr, msg_e = _edge1_call(gd, gs, eattr, p1['weT'],
                               p1['eu2T'], p1['eu2b'], p1['f1T'], p1['f1b'],
                               p1['f2T'], p1['f2b'])
    msgp = _scatter_call(msg_e, dst, zer)
    h2, pd, ps = _node_call(msgp, h1, p1['sm1T'], p1['sm1b'], p1['sm2T'],
                            p1['sm2b'], p2['wiT'], p2['b1'], p2['wjT'],
                            p2['cfT'])

    gd, gs = _gather_tables(pd, dst, ps, src)
    msg_e = _edge2_call(gd, gs, eattr, p2['weT'],
                        p2['wcT'], p2['bc'], p2['f2T'], p2['f2b'])
    msgp = _scatter_call(msg_e, dst, zer)

    lin1T = params['lin1_W'].T
    lin2T = params['lin2_W'].T
    out = _final_call(msgp, h2, p2['sm1T'], p2['sm1b'], p2['sm2T'],
                      p2['sm2b'], lin1T,
                      params['lin1_b'].reshape(1, H // 2), lin2T,
                      params['lin2_b'].reshape(1, 1), batch2)
    return out


# speedup vs baseline: 3.2563x; 1.1891x over previous
import functools

import jax
import jax.numpy as jnp
import numpy as np
from jax import lax
from jax.experimental import pallas as pl
from jax.experimental.pallas import tpu as pltpu
from jax.experimental.pallas import tpu_sc as plsc

N = 10000
E = 160000
H = 128
F = 128
NG = 150
NGP = 160
NI = 3
NUM_EMB = 100
NUM_EMB_P = 128
CUTOFF = 15.0
NG_GRAPHS = 512
LOG2 = float(np.log(2.0))

TILE_N = 2000
TILE_E = 4000

NW = 32
EA = 96000
EB = E - EA
CH = 40
CH2 = 40
NP = 10240
RPT = NP // 16


_LOG2E = 1.4426950408889634


def _ssp(x):
    t = jnp.exp2(jnp.abs(x) * (-_LOG2E))
    return jnp.maximum(x, 0.0) + jnp.log2(1.0 + t) * LOG2 - LOG2


_sspb = _ssp


def _f32(shape):
    return jax.ShapeDtypeStruct(shape, jnp.float32)


def _bf16(shape):
    return jax.ShapeDtypeStruct(shape, jnp.bfloat16)


_MASKHI = -65536


def _pack2(a, b):
    ab = lax.bitcast_convert_type(
        a.astype(jnp.bfloat16).astype(jnp.float32), jnp.int32)
    bb = lax.bitcast_convert_type(
        b.astype(jnp.bfloat16).astype(jnp.float32), jnp.int32)
    word = jnp.bitwise_or(lax.shift_right_logical(ab, 16),
                          jnp.bitwise_and(bb, _MASKHI))
    return lax.bitcast_convert_type(word, jnp.float32)


def _unpack2(p):
    u = lax.bitcast_convert_type(p, jnp.int32)
    a = lax.bitcast_convert_type(lax.shift_left(u, 16), jnp.float32)
    b = lax.bitcast_convert_type(jnp.bitwise_and(u, _MASKHI), jnp.float32)
    return a, b


def _unpack_cat(p):
    a, b = _unpack2(p)
    return jnp.concatenate([a, b], axis=1)


def _full(shape):
    return pl.BlockSpec(shape, lambda i: (0,) * len(shape))


def _dot(a, b):
    return jnp.dot(a, b, preferred_element_type=jnp.float32)


def _bdot(a, b_ref):
    return jnp.dot(a.astype(jnp.bfloat16), b_ref[...],
                   preferred_element_type=jnp.float32)


def _init_body(z_ref, pos_ref, emb_ref, wiT_ref, b1_ref, wjT_ref, cfT_ref,
               h_ref, pdst_ref, psrc_ref):
    zv = z_ref[...]
    oh = (zv == lax.broadcasted_iota(jnp.int32, (TILE_N, NUM_EMB_P), 1))
    h = _dot(oh.astype(jnp.float32), emb_ref[...])
    h_ref[...] = h
    pp = pos_ref[...]
    vi = _dot(h, wiT_ref[...]) + b1_ref[...]
    vj = _dot(h, wjT_ref[...])
    cf = _dot(h, cfT_ref[...])
    pdst_ref[...] = jnp.concatenate(
        [_pack2(vi[:, :H], vi[:, H:]), pp], axis=1)
    psrc_ref[...] = jnp.concatenate(
        [_pack2(vj[:, :H], vj[:, H:]),
         _pack2(cf[:, :H // 2], cf[:, H // 2:]),
         pp[:, :H // 2]], axis=1)


_init_call = pl.pallas_call(
    _init_body,
    grid=(N // TILE_N,),
    in_specs=[
        pl.BlockSpec((TILE_N, 1), lambda i: (i, 0)),
        pl.BlockSpec((TILE_N, H), lambda i: (i, 0)),
        _full((NUM_EMB_P, H)),
        _full((H, 2 * H)),
        _full((1, 2 * H)),
        _full((H, 2 * H)),
        _full((H, H)),
    ],
    out_specs=[
        pl.BlockSpec((TILE_N, H), lambda i: (i, 0)),
        pl.BlockSpec((TILE_N, 2 * H), lambda i: (i, 0)),
        pl.BlockSpec((TILE_N, 2 * H), lambda i: (i, 0)),
    ],
    out_shape=[_f32((N, H)), _f32((N, 2 * H)), _f32((N, 2 * H))],
)


def _node_body(msgpa_ref, msgpb_ref, h_ref, sm1T_ref, sm1b_ref, sm2T_ref,
               sm2b_ref, wiT_ref, b1_ref, wjT_ref, cfT_ref,
               hout_ref, pdst_ref, psrc_ref):
    ma = msgpa_ref[...]
    mb = msgpb_ref[...]
    msg = (ma[0] + ma[1]) + (mb[0] + mb[1])
    u = _ssp(_dot(msg, sm1T_ref[...]) + sm1b_ref[...])
    h2 = h_ref[...] + _dot(u, sm2T_ref[...]) + sm2b_ref[...]
    hout_ref[...] = h2
    vi = _dot(h2, wiT_ref[...]) + b1_ref[...]
    vj = _dot(h2, wjT_ref[...])
    cf = _dot(h2, cfT_ref[...])
    pdst_ref[...] = _pack2(vi[:, :H], vi[:, H:])
    psrc_ref[...] = jnp.concatenate([_pack2(vj[:, :H], vj[:, H:]), cf],
                                    axis=1)


_node_call = pl.pallas_call(
    _node_body,
    grid=(N // TILE_N,),
    in_specs=[
        pl.BlockSpec((2, TILE_N, H), lambda i: (0, i, 0)),
        pl.BlockSpec((2, TILE_N, H), lambda i: (0, i, 0)),
        pl.BlockSpec((TILE_N, H), lambda i: (i, 0)),
        _full((H, H)), _full((1, H)), _full((H, H)), _full((1, H)),
        _full((H, 2 * H)), _full((1, 2 * H)), _full((H, 2 * H)),
        _full((H, H)),
    ],
    out_specs=[
        pl.BlockSpec((TILE_N, H), lambda i: (i, 0)),
        pl.BlockSpec((TILE_N, H), lambda i: (i, 0)),
        pl.BlockSpec((TILE_N, 2 * H), lambda i: (i, 0)),
    ],
    out_shape=[_f32((N, H)), _f32((N, H)), _f32((N, 2 * H))],
)


def _final_body(msgpa_ref, msgpb_ref, h_ref, sm1T_ref, sm1b_ref, sm2T_ref,
                sm2b_ref, l1T_ref, l1b_ref, l2T_ref, l2b_ref, batch_ref,
                out_ref):
    i = pl.program_id(0)
    ma = msgpa_ref[...]
    mb = msgpb_ref[...]
    msg = (ma[0] + ma[1]) + (mb[0] + mb[1])
    u = _ssp(_dot(msg, sm1T_ref[...]) + sm1b_ref[...])
    h2 = h_ref[...] + _dot(u, sm2T_ref[...]) + sm2b_ref[...]
    no = _dot(_ssp(_dot(h2, l1T_ref[...]) + l1b_ref[...]), l2T_ref[...])
    no = no + l2b_ref[...]
    b2 = batch_ref[0]
    mask = (lax.broadcasted_iota(jnp.int32, (NG_GRAPHS, TILE_N), 0) == b2)
    part = _dot(mask.astype(jnp.float32), no)

    @pl.when(i == 0)
    def _zero():
        out_ref[...] = jnp.zeros_like(out_ref)

    out_ref[...] += part


_final_call = pl.pallas_call(
    _final_body,
    grid=(N // TILE_N,),
    in_specs=[
        pl.BlockSpec((2, TILE_N, H), lambda i: (0, i, 0)),
        pl.BlockSpec((2, TILE_N, H), lambda i: (0, i, 0)),
        pl.BlockSpec((TILE_N, H), lambda i: (i, 0)),
        _full((H, H)), _full((1, H)), _full((H, H)), _full((1, H)),
        _full((H, H // 2)), _full((1, H // 2)), _full((H // 2, 1)),
        _full((1, 1)),
        pl.BlockSpec((1, 1, TILE_N), lambda i: (i, 0, 0)),
    ],
    out_specs=pl.BlockSpec((NG_GRAPHS, 1), lambda i: (0, 0)),
    out_shape=_f32((NG_GRAPHS, 1)),
)


_COEFF = -0.5 / (CUTOFF / NG) ** 2


def _edge0_body(gdst_ref, gsrc_ref, offs_ref, weT_ref,
                eu2T_ref, eu2b_ref, f1T_ref, f1b_ref, f2T_ref, f2b_ref,
                eattr_ref, msg_ref):
    gd = gdst_ref[...]
    s = gsrc_ref[...]
    d = gd[:, H:H + H // 2] - s[:, H + H // 2:]
    dist = jnp.sqrt(jnp.sum(d * d, axis=1, keepdims=True))
    dif = dist - offs_ref[...]
    g = jnp.exp2((dif * dif * (_COEFF * _LOG2E)).astype(jnp.bfloat16))
    pd = _unpack_cat(gd[:, :H])
    pj = _unpack_cat(s[:, :H])
    cf = _unpack_cat(s[:, H:H + H // 2])
    ea = _sspb(_bdot(g, weT_ref) + pd + pj)
    eattr = _bdot(ea, eu2T_ref) + eu2b_ref[...]
    eattr_ref[...] = eattr.astype(jnp.bfloat16)
    w = _sspb(_bdot(eattr, f1T_ref) + f1b_ref[...])
    w = _sspb(_bdot(w, f2T_ref) + f2b_ref[...])
    msg_ref[...] = cf * w.astype(jnp.float32)


def _make_edge0(ne):
    return pl.pallas_call(
        _edge0_body,
        grid=(ne // TILE_E,),
        in_specs=[
            pl.BlockSpec((TILE_E, 2 * H), lambda i: (i, 0)),
            pl.BlockSpec((TILE_E, 2 * H), lambda i: (i, 0)),
            _full((1, NGP)), _full((NGP, 2 * H)),
            _full((2 * H, H)), _full((1, H)),
            _full((H, H)), _full((1, H)), _full((H, H)), _full((1, H)),
        ],
        out_specs=[
            pl.BlockSpec((TILE_E, H), lambda i: (i, 0)),
            pl.BlockSpec((TILE_E, H), lambda i: (i, 0)),
        ],
        out_shape=[_bf16((ne, H)), _f32((ne, H))],
    )


def _edge1_body(gdst_ref, gsrc_ref, eain_ref, weT_ref,
                eu2T_ref, eu2b_ref, f1T_ref, f1b_ref, f2T_ref, f2b_ref,
                eattr_ref, msg_ref):
    s = gsrc_ref[...]
    pd = _unpack_cat(gdst_ref[...])
    pj = _unpack_cat(s[:, :H])
    eterm = jnp.dot(eain_ref[...], weT_ref[...],
                    preferred_element_type=jnp.float32)
    ea = _sspb(eterm + pd + pj)
    eattr = _bdot(ea, eu2T_ref) + eu2b_ref[...]
    eattr_ref[...] = eattr.astype(jnp.bfloat16)
    w = _sspb(_bdot(eattr, f1T_ref) + f1b_ref[...])
    w = _sspb(_bdot(w, f2T_ref) + f2b_ref[...])
    msg_ref[...] = s[:, H:] * w.astype(jnp.float32)


def _make_edge1(ne):
    return pl.pallas_call(
        _edge1_body,
        grid=(ne // TILE_E,),
        in_specs=[
            pl.BlockSpec((TILE_E, H), lambda i: (i, 0)),
            pl.BlockSpec((TILE_E, 2 * H), lambda i: (i, 0)),
            pl.BlockSpec((TILE_E, H), lambda i: (i, 0)),
            _full((H, 2 * H)),
            _full((2 * H, H)), _full((1, H)),
            _full((H, H)), _full((1, H)), _full((H, H)), _full((1, H)),
        ],
        out_specs=[
            pl.BlockSpec((TILE_E, H), lambda i: (i, 0)),
            pl.BlockSpec((TILE_E, H), lambda i: (i, 0)),
        ],
        out_shape=[_bf16((ne, H)), _f32((ne, H))],
    )


def _edge2_body(gdst_ref, gsrc_ref, eain_ref, weT_ref,
                wcT_ref, bc_ref, f2T_ref, f2b_ref, msg_ref):
    s = gsrc_ref[...]
    pd = _unpack_cat(gdst_ref[...])
    pj = _unpack_cat(s[:, :H])
    eterm = jnp.dot(eain_ref[...], weT_ref[...],
                    preferred_element_type=jnp.float32)
    ea = _sspb(eterm + pd + pj)
    w = _sspb(_bdot(ea, wcT_ref) + bc_ref[...])
    w = _sspb(_bdot(w, f2T_ref) + f2b_ref[...])
    msg_ref[...] = s[:, H:] * w.astype(jnp.float32)


def _make_edge2(ne):
    return pl.pallas_call(
        _edge2_body,
        grid=(ne // TILE_E,),
        in_specs=[
            pl.BlockSpec((TILE_E, H), lambda i: (i, 0)),
            pl.BlockSpec((TILE_E, 2 * H), lambda i: (i, 0)),
            pl.BlockSpec((TILE_E, H), lambda i: (i, 0)),
            _full((H, 2 * H)),
            _full((2 * H, H)), _full((1, H)),
            _full((H, H)), _full((1, H)),
        ],
        out_specs=pl.BlockSpec((TILE_E, H), lambda i: (i, 0)),
        out_shape=_f32((ne, H)),
    )


_edge0_A, _edge0_B = _make_edge0(EA), _make_edge0(EB)
_edge1_A, _edge1_B = _make_edge1(EA), _make_edge1(EB)
_edge2_A, _edge2_B = _make_edge2(EA), _make_edge2(EB)


_MESH = plsc.VectorSubcoreMesh(core_axis_name="c", subcore_axis_name="s")


def _make_gather2(d1, d2, ne):
    epw = ne // NW
    assert epw % CH == 0 and epw % 8 == 0
    nch = epw // CH

    @functools.partial(
        pl.kernel,
        out_type=(_f32((ne, d1)), _f32((ne, d2))),
        mesh=_MESH,
        scratch_types=[
            pltpu.VMEM((CH,), jnp.int32), pltpu.VMEM((CH,), jnp.int32),
            pltpu.VMEM((CH,), jnp.int32), pltpu.VMEM((CH,), jnp.int32),
            pltpu.VMEM((CH, d1), jnp.float32),
            pltpu.VMEM((CH, d1), jnp.float32),
            pltpu.VMEM((CH, d2), jnp.float32),
            pltpu.VMEM((CH, d2), jnp.float32),
            pltpu.SemaphoreType.DMA, pltpu.SemaphoreType.DMA,
            pltpu.SemaphoreType.DMA, pltpu.SemaphoreType.DMA,
        ],
    )
    def gath(t1_hbm, i1_hbm, t2_hbm, i2_hbm, o1_hbm, o2_hbm,
             ia0, ia1, ib0, ib1, ra0, ra1, rb0, rb1, sa0, sa1, sb0, sb1):
        ia, ib = (ia0, ia1), (ib0, ib1)
        ra, rb = (ra0, ra1), (rb0, rb1)
        sa, sb = (sa0, sa1), (sb0, sb1)
        wid = lax.axis_index("s") * 2 + lax.axis_index("c")
        base = wid * epw

        def load_start(buf, off):
            pltpu.sync_copy(i1_hbm.at[pl.ds(off, CH)], ia[buf])
            pltpu.sync_copy(i2_hbm.at[pl.ds(off, CH)], ib[buf])
            pltpu.async_copy(t1_hbm.at[ia[buf]], ra[buf], sa[buf])
            pltpu.async_copy(t2_hbm.at[ib[buf]], rb[buf], sb[buf])

        def wait_store(buf, off):
            pltpu.make_async_copy(t1_hbm.at[ia[buf]], ra[buf], sa[buf]).wait()
            pltpu.make_async_copy(t2_hbm.at[ib[buf]], rb[buf], sb[buf]).wait()
            pltpu.sync_copy(ra[buf], o1_hbm.at[pl.ds(off, CH)])
            pltpu.sync_copy(rb[buf], o2_hbm.at[pl.ds(off, CH)])

        load_start(0, base)

        @pl.loop(0, 2 * ((nch - 1) // 2), step=2)
        def _pair(ci):
            for b in range(2):
                off = base + (ci + b) * CH
                load_start(1 - b, off + CH)
                wait_store(b, off)

        if nch % 2 == 1:
            wait_store(0, base + (nch - 1) * CH)
        else:
            load_start(1, base + (nch - 1) * CH)
            wait_store(0, base + (nch - 2) * CH)
            wait_store(1, base + (nch - 1) * CH)

    return gath


_gather_A = _make_gather2(H, 2 * H, EA)
_gather_B = _make_gather2(H, 2 * H, EB)
_gather0_A = _make_gather2(2 * H, 2 * H, EA)
_gather0_B = _make_gather2(2 * H, 2 * H, EB)


def _make_scatter(ne):
    epw = ne // NW
    assert epw % CH2 == 0 and epw % 8 == 0
    nch = epw // CH2

    @functools.partial(
        pl.kernel,
        out_type=_f32((2, NP, H)),
        mesh=_MESH,
        scratch_types=[
            pltpu.VMEM((CH2,), jnp.int32), pltpu.VMEM((CH2,), jnp.int32),
            pltpu.VMEM((CH2, H), jnp.float32),
            pltpu.VMEM((CH2, H), jnp.float32),
            pltpu.VMEM_SHARED((NP, H), jnp.float32),
            pltpu.SemaphoreType.DMA, pltpu.SemaphoreType.DMA,
        ],
    )
    def scat(msg_hbm, dst_hbm, zer_hbm, out_hbm,
             i0, i1, r0, r1, acc, s0, s1):
        c = lax.axis_index("c")
        s = lax.axis_index("s")
        wid = s * 2 + c
        idx, rows, sem = (i0, i1), (r0, r1), (s0, s1)
        pltpu.sync_copy(zer_hbm, acc.at[pl.ds(s * RPT, RPT)])
        plsc.subcore_barrier()
        base = wid * epw

        def load_start(buf, off):
            pltpu.sync_copy(dst_hbm.at[pl.ds(off, CH2)], idx[buf])
            pltpu.async_copy(msg_hbm.at[pl.ds(off, CH2)], rows[buf],
                             sem[buf])

        def wait_add(buf, off):
            pltpu.make_async_copy(msg_hbm.at[pl.ds(off, CH2)], rows[buf],
                                  sem[buf]).wait()
            pltpu.sync_copy(rows[buf], acc.at[idx[buf]], add=True)

        load_start(0, base)

        @pl.loop(0, 2 * ((nch - 1) // 2), step=2)
        def _pair(ci):
            for b in range(2):
                off = base + (ci + b) * CH2
                load_start(1 - b, off + CH2)
                wait_add(b, off)

        if nch % 2 == 1:
            wait_add(0, base + (nch - 1) * CH2)
        else:
            load_start(1, base + (nch - 1) * CH2)
            wait_add(0, base + (nch - 2) * CH2)
            wait_add(1, base + (nch - 1) * CH2)
        plsc.subcore_barrier()
        pltpu.sync_copy(acc.at[pl.ds(s * RPT, RPT)],
                        out_hbm.at[c, pl.ds(s * RPT, RPT)])

    return scat


_scatter_A = _make_scatter(EA)
_scatter_B = _make_scatter(EB)


def kernel(z, pos, edge_index, batch, params):
    src = edge_index[0].astype(jnp.int32)
    dst = edge_index[1].astype(jnp.int32)
    z2 = z.astype(jnp.int32).reshape(N, 1)
    batch2 = batch.astype(jnp.int32).reshape(N // TILE_N, 1, TILE_N)
    pospad = jnp.pad(pos.astype(jnp.float32), ((0, 0), (0, H - 3)))
    zer = jnp.zeros((RPT, H), jnp.float32)

    stop = CUTOFF - CUTOFF / NG
    offs = jnp.pad(jnp.linspace(0.0, stop, NG, dtype=jnp.float32),
                   (0, NGP - NG)).reshape(1, NGP)

    emb = jnp.pad(params['embedding'], ((0, NUM_EMB_P - NUM_EMB), (0, 0)))

    prep = []
    for t, p in enumerate(params['interactions']):
        w1 = p['eu1_W']
        wiT = w1[:, :H].T
        wjT = w1[:, H:2 * H].T
        weT = w1[:, 2 * H:].T
        if t == 0:
            weT = jnp.pad(weT, ((0, NGP - NG), (0, 0)))
        d = dict(
            wiT=wiT, b1=p['eu1_b'].reshape(1, 2 * H), wjT=wjT,
            cfT=p['cf_W'].T,
            weT=weT.astype(jnp.bfloat16),
            eu2T=p['eu2_W'].T.astype(jnp.bfloat16),
            eu2b=p['eu2_b'].reshape(1, H),
            f1T=p['f1_W'].T.astype(jnp.bfloat16),
            f1b=p['f1_b'].reshape(1, H),
            f2T=p['f2_W'].T.astype(jnp.bfloat16),
            f2b=p['f2_b'].reshape(1, H),
            sm1T=p['sm1_W'].T, sm1b=p['sm1_b'].reshape(1, H),
            sm2T=p['sm2_W'].T, sm2b=p['sm2_b'].reshape(1, H),
        )
        if t == NI - 1:
            d['wcT'] = (p['eu2_W'].T @ p['f1_W'].T).astype(jnp.bfloat16)
            d['bc'] = (p['eu2_b'] @ p['f1_W'].T
                       + p['f1_b']).reshape(1, H)
        prep.append(d)

    p0, p1, p2 = prep

    srcA, srcB = src[:EA], src[EA:]
    dstA, dstB = dst[:EA], dst[EA:]

    h0, pd, ps = _init_call(z2, pospad, emb, p0['wiT'], p0['b1'], p0['wjT'],
                            p0['cfT'])

    gdA, gsA = _gather0_A(pd, dstA, ps, srcA)
    gdB, gsB = _gather0_B(pd, dstB, ps, srcB)
    ea = (offs, p0['weT'], p0['eu2T'], p0['eu2b'], p0['f1T'], p0['f1b'],
          p0['f2T'], p0['f2b'])
    eattrA, msgA = _edge0_A(gdA, gsA, *ea)
    eattrB, msgB = _edge0_B(gdB, gsB, *ea)
    mpA = _scatter_A(msgA, dstA, zer)
    mpB = _scatter_B(msgB, dstB, zer)
    h1, pd, ps = _node_call(mpA, mpB, h0, p0['sm1T'], p0['sm1b'],
                            p0['sm2T'], p0['sm2b'], p1['wiT'], p1['b1'],
                            p1['wjT'], p1['cfT'])

    gdA, gsA = _gather_A(pd, dstA, ps, srcA)
    gdB, gsB = _gather_B(pd, dstB, ps, srcB)
    ea = (p1['weT'], p1['eu2T'], p1['eu2b'], p1['f1T'], p1['f1b'],
          p1['f2T'], p1['f2b'])
    eattrA, msgA = _edge1_A(gdA, gsA, eattrA, *ea)
    eattrB, msgB = _edge1_B(gdB, gsB, eattrB, *ea)
    mpA = _scatter_A(msgA, dstA, zer)
    mpB = _scatter_B(msgB, dstB, zer)
    h2, pd, ps = _node_call(mpA, mpB, h1, p1['sm1T'], p1['sm1b'],
                            p1['sm2T'], p1['sm2b'], p2['wiT'], p2['b1'],
                            p2['wjT'], p2['cfT'])

    gdA, gsA = _gather_A(pd, dstA, ps, srcA)
    gdB, gsB = _gather_B(pd, dstB, ps, srcB)
    ea = (p2['weT'], p2['wcT'], p2['bc'], p2['f2T'], p2['f2b'])
    msgA = _edge2_A(gdA, gsA, eattrA, *ea)
    msgB = _edge2_B(gdB, gsB, eattrB, *ea)
    mpA = _scatter_A(msgA, dstA, zer)
    mpB = _scatter_B(msgB, dstB, zer)

    lin1T = params['lin1_W'].T
    lin2T = params['lin2_W'].T
    out = _final_call(mpA, mpB, h2, p2['sm1T'], p2['sm1b'], p2['sm2T'],
                      p2['sm2b'], lin1T,
                      params['lin1_b'].reshape(1, H // 2), lin2T,
                      params['lin2_b'].reshape(1, 1), batch2)
    return out


# speedup vs baseline: 3.5548x; 1.0917x over previous
import functools

import jax
import jax.numpy as jnp
import numpy as np
from jax import lax
from jax.experimental import pallas as pl
from jax.experimental.pallas import tpu as pltpu
from jax.experimental.pallas import tpu_sc as plsc

N = 10000
E = 160000
H = 128
F = 128
NG = 150
NGP = 160
NI = 3
NUM_EMB = 100
NUM_EMB_P = 128
CUTOFF = 15.0
NG_GRAPHS = 512
LOG2 = float(np.log(2.0))

TILE_N = 2000
TILE_E = 4000

NW = 32
EA = 96000
EB = E - EA
CH = 40
CH2 = 40
NP = 10240
RPT = NP // 16


_LOG2E = 1.4426950408889634


def _ssp(x):
    t = jnp.exp2(jnp.abs(x) * (-_LOG2E))
    return jnp.maximum(x, 0.0) + jnp.log2(1.0 + t) * LOG2 - LOG2


_sspb = _ssp


def _f32(shape):
    return jax.ShapeDtypeStruct(shape, jnp.float32)


def _bf16(shape):
    return jax.ShapeDtypeStruct(shape, jnp.bfloat16)


_MASKHI = -65536


def _pack2(a, b):
    ab = lax.bitcast_convert_type(
        a.astype(jnp.bfloat16).astype(jnp.float32), jnp.int32)
    bb = lax.bitcast_convert_type(
        b.astype(jnp.bfloat16).astype(jnp.float32), jnp.int32)
    word = jnp.bitwise_or(lax.shift_right_logical(ab, 16),
                          jnp.bitwise_and(bb, _MASKHI))
    return lax.bitcast_convert_type(word, jnp.float32)


def _unpack2(p):
    u = lax.bitcast_convert_type(p, jnp.int32)
    a = lax.bitcast_convert_type(lax.shift_left(u, 16), jnp.float32)
    b = lax.bitcast_convert_type(jnp.bitwise_and(u, _MASKHI), jnp.float32)
    return a, b


def _unpack_cat(p):
    a, b = _unpack2(p)
    return jnp.concatenate([a, b], axis=1)


def _full(shape):
    return pl.BlockSpec(shape, lambda i: (0,) * len(shape))


def _dot(a, b):
    return jnp.dot(a, b, preferred_element_type=jnp.float32)


def _bdot(a, b_ref):
    return jnp.dot(a.astype(jnp.bfloat16), b_ref[...],
                   preferred_element_type=jnp.float32)


def _init_body(z_ref, pos_ref, emb_ref, wiT_ref, b1_ref, wjT_ref, cfT_ref,
               h_ref, pdst_ref, psrc_ref):
    zv = z_ref[...]
    oh = (zv == lax.broadcasted_iota(jnp.int32, (TILE_N, NUM_EMB_P), 1))
    h = _dot(oh.astype(jnp.float32), emb_ref[...])
    h_ref[...] = h
    pp = pos_ref[...]
    vi = _dot(h, wiT_ref[...]) + b1_ref[...]
    vj = _dot(h, wjT_ref[...])
    cf = _dot(h, cfT_ref[...])
    pdst_ref[...] = jnp.concatenate(
        [_pack2(vi[:, :H], vi[:, H:]), pp], axis=1)
    psrc_ref[...] = jnp.concatenate(
        [_pack2(vj[:, :H], vj[:, H:]),
         _pack2(cf[:, :H // 2], cf[:, H // 2:]),
         pp[:, :H // 2]], axis=1)


_init_call = pl.pallas_call(
    _init_body,
    grid=(N // TILE_N,),
    in_specs=[
        pl.BlockSpec((TILE_N, 1), lambda i: (i, 0)),
        pl.BlockSpec((TILE_N, H), lambda i: (i, 0)),
        _full((NUM_EMB_P, H)),
        _full((H, 2 * H)),
        _full((1, 2 * H)),
        _full((H, 2 * H)),
        _full((H, H)),
    ],
    out_specs=[
        pl.BlockSpec((TILE_N, H), lambda i: (i, 0)),
        pl.BlockSpec((TILE_N, 2 * H), lambda i: (i, 0)),
        pl.BlockSpec((TILE_N, 2 * H), lambda i: (i, 0)),
    ],
    out_shape=[_f32((N, H)), _f32((N, 2 * H)), _f32((N, 2 * H))],
)


def _node_body(msgpa_ref, msgpb_ref, h_ref, sm1T_ref, sm1b_ref, sm2T_ref,
               sm2b_ref, wiT_ref, b1_ref, wjT_ref, cfT_ref,
               hout_ref, pdst_ref, psrc_ref):
    ma = msgpa_ref[...]
    mb = msgpb_ref[...]
    msg = (ma[0] + ma[1]) + (mb[0] + mb[1])
    u = _ssp(_dot(msg, sm1T_ref[...]) + sm1b_ref[...])
    h2 = h_ref[...] + _dot(u, sm2T_ref[...]) + sm2b_ref[...]
    hout_ref[...] = h2
    vi = _dot(h2, wiT_ref[...]) + b1_ref[...]
    vj = _dot(h2, wjT_ref[...])
    cf = _dot(h2, cfT_ref[...])
    pdst_ref[...] = _pack2(vi[:, :H], vi[:, H:])
    psrc_ref[...] = jnp.concatenate([_pack2(vj[:, :H], vj[:, H:]), cf],
                                    axis=1)


_node_call = pl.pallas_call(
    _node_body,
    grid=(N // TILE_N,),
    in_specs=[
        pl.BlockSpec((2, TILE_N, H), lambda i: (0, i, 0)),
        pl.BlockSpec((2, TILE_N, H), lambda i: (0, i, 0)),
        pl.BlockSpec((TILE_N, H), lambda i: (i, 0)),
        _full((H, H)), _full((1, H)), _full((H, H)), _full((1, H)),
        _full((H, 2 * H)), _full((1, 2 * H)), _full((H, 2 * H)),
        _full((H, H)),
    ],
    out_specs=[
        pl.BlockSpec((TILE_N, H), lambda i: (i, 0)),
        pl.BlockSpec((TILE_N, H), lambda i: (i, 0)),
        pl.BlockSpec((TILE_N, 2 * H), lambda i: (i, 0)),
    ],
    out_shape=[_f32((N, H)), _f32((N, H)), _f32((N, 2 * H))],
)


def _final_body(msgpa_ref, msgpb_ref, h_ref, sm1T_ref, sm1b_ref, sm2T_ref,
                sm2b_ref, l1T_ref, l1b_ref, l2T_ref, l2b_ref, batch_ref,
                out_ref):
    i = pl.program_id(0)
    ma = msgpa_ref[...]
    mb = msgpb_ref[...]
    msg = (ma[0] + ma[1]) + (mb[0] + mb[1])
    u = _ssp(_dot(msg, sm1T_ref[...]) + sm1b_ref[...])
    h2 = h_ref[...] + _dot(u, sm2T_ref[...]) + sm2b_ref[...]
    no = _dot(_ssp(_dot(h2, l1T_ref[...]) + l1b_ref[...]), l2T_ref[...])
    no = no + l2b_ref[...]
    b2 = batch_ref[0]
    mask = (lax.broadcasted_iota(jnp.int32, (NG_GRAPHS, TILE_N), 0) == b2)
    part = _dot(mask.astype(jnp.float32), no)

    @pl.when(i == 0)
    def _zero():
        out_ref[...] = jnp.zeros_like(out_ref)

    out_ref[...] += part


_final_call = pl.pallas_call(
    _final_body,
    grid=(N // TILE_N,),
    in_specs=[
        pl.BlockSpec((2, TILE_N, H), lambda i: (0, i, 0)),
        pl.BlockSpec((2, TILE_N, H), lambda i: (0, i, 0)),
        pl.BlockSpec((TILE_N, H), lambda i: (i, 0)),
        _full((H, H)), _full((1, H)), _full((H, H)), _full((1, H)),
        _full((H, H // 2)), _full((1, H // 2)), _full((H // 2, 1)),
        _full((1, 1)),
        pl.BlockSpec((1, 1, TILE_N), lambda i: (i, 0, 0)),
    ],
    out_specs=pl.BlockSpec((NG_GRAPHS, 1), lambda i: (0, 0)),
    out_shape=_f32((NG_GRAPHS, 1)),
)


_COEFF = -0.5 / (CUTOFF / NG) ** 2


def _edge0_body(gdst_ref, gsrc_ref, offs_ref, weT_ref,
                eu2T_ref, eu2b_ref, f1T_ref, f1b_ref, f2T_ref, f2b_ref,
                eattr_ref, msg_ref):
    gd = gdst_ref[...]
    s = gsrc_ref[...]
    d = gd[:, H:H + H // 2] - s[:, H + H // 2:]
    dist = jnp.sqrt(jnp.sum(d * d, axis=1, keepdims=True))
    dif = dist - offs_ref[...]
    g = jnp.exp2((dif * dif * (_COEFF * _LOG2E)).astype(jnp.bfloat16))
    pd = _unpack_cat(gd[:, :H])
    pj = _unpack_cat(s[:, :H])
    cf = _unpack_cat(s[:, H:H + H // 2])
    ea = _sspb(_bdot(g, weT_ref) + pd + pj)
    eattr = _bdot(ea, eu2T_ref) + eu2b_ref[...]
    eattr_ref[...] = eattr.astype(jnp.bfloat16)
    w = _sspb(_bdot(eattr, f1T_ref) + f1b_ref[...])
    w = _sspb(_bdot(w, f2T_ref) + f2b_ref[...])
    msg_ref[...] = cf * w.astype(jnp.float32)


def _make_edge0(ne):
    return pl.pallas_call(
        _edge0_body,
        grid=(ne // TILE_E,),
        in_specs=[
            pl.BlockSpec((TILE_E, 2 * H), lambda i: (i, 0)),
            pl.BlockSpec((TILE_E, 2 * H), lambda i: (i, 0)),
            _full((1, NGP)), _full((NGP, 2 * H)),
            _full((2 * H, H)), _full((1, H)),
            _full((H, H)), _full((1, H)), _full((H, H)), _full((1, H)),
        ],
        out_specs=[
            pl.BlockSpec((TILE_E, H), lambda i: (i, 0)),
            pl.BlockSpec((TILE_E, H), lambda i: (i, 0)),
        ],
        out_shape=[_bf16((ne, H)), _f32((ne, H))],
    )


def _edge1_body(gdst_ref, gsrc_ref, eain_ref, weT_ref,
                eu2T_ref, eu2b_ref, f1T_ref, f1b_ref, f2T_ref, f2b_ref,
                eattr_ref, msg_ref):
    s = gsrc_ref[...]
    pd = _unpack_cat(gdst_ref[...])
    pj = _unpack_cat(s[:, :H])
    eterm = jnp.dot(eain_ref[...], weT_ref[...],
                    preferred_element_type=jnp.float32)
    ea = _sspb(eterm + pd + pj)
    eattr = _bdot(ea, eu2T_ref) + eu2b_ref[...]
    eattr_ref[...] = eattr.astype(jnp.bfloat16)
    w = _sspb(_bdot(eattr, f1T_ref) + f1b_ref[...])
    w = _sspb(_bdot(w, f2T_ref) + f2b_ref[...])
    msg_ref[...] = s[:, H:] * w.astype(jnp.float32)


def _make_edge1(ne):
    return pl.pallas_call(
        _edge1_body,
        grid=(ne // TILE_E,),
        in_specs=[
            pl.BlockSpec((TILE_E, H), lambda i: (i, 0)),
            pl.BlockSpec((TILE_E, 2 * H), lambda i: (i, 0)),
            pl.BlockSpec((TILE_E, H), lambda i: (i, 0)),
            _full((H, 2 * H)),
            _full((2 * H, H)), _full((1, H)),
            _full((H, H)), _full((1, H)), _full((H, H)), _full((1, H)),
        ],
        out_specs=[
            pl.BlockSpec((TILE_E, H), lambda i: (i, 0)),
            pl.BlockSpec((TILE_E, H), lambda i: (i, 0)),
        ],
        out_shape=[_bf16((ne, H)), _f32((ne, H))],
    )


def _edge2_body(gdst_ref, gsrc_ref, eain_ref, weT_ref,
                wcT_ref, bc_ref, f2T_ref, f2b_ref, msg_ref):
    s = gsrc_ref[...]
    pd = _unpack_cat(gdst_ref[...])
    pj = _unpack_cat(s[:, :H])
    eterm = jnp.dot(eain_ref[...], weT_ref[...],
                    preferred_element_type=jnp.float32)
    ea = _sspb(eterm + pd + pj)
    w = _sspb(_bdot(ea, wcT_ref) + bc_ref[...])
    w = _sspb(_bdot(w, f2T_ref) + f2b_ref[...])
    msg_ref[...] = s[:, H:] * w.astype(jnp.float32)


def _make_edge2(ne):
    return pl.pallas_call(
        _edge2_body,
        grid=(ne // TILE_E,),
        in_specs=[
            pl.BlockSpec((TILE_E, H), lambda i: (i, 0)),
            pl.BlockSpec((TILE_E, 2 * H), lambda i: (i, 0)),
            pl.BlockSpec((TILE_E, H), lambda i: (i, 0)),
            _full((H, 2 * H)),
            _full((2 * H, H)), _full((1, H)),
            _full((H, H)), _full((1, H)),
        ],
        out_specs=pl.BlockSpec((TILE_E, H), lambda i: (i, 0)),
        out_shape=_f32((ne, H)),
    )


_edge0_A, _edge0_B = _make_edge0(EA), _make_edge0(EB)
_edge1_A, _edge1_B = _make_edge1(EA), _make_edge1(EB)
_edge2_A, _edge2_B = _make_edge2(EA), _make_edge2(EB)


_MESH = plsc.VectorSubcoreMesh(core_axis_name="c", subcore_axis_name="s")


def _make_gather2(d1, d2, ne, ch):
    CH = ch
    epw = ne // NW
    assert epw % CH == 0 and epw % 8 == 0 and CH % 8 == 0 and CH <= 128
    nch = epw // CH

    @functools.partial(
        pl.kernel,
        out_type=(_f32((ne, d1)), _f32((ne, d2))),
        mesh=_MESH,
        scratch_types=[
            pltpu.VMEM((CH,), jnp.int32), pltpu.VMEM((CH,), jnp.int32),
            pltpu.VMEM((CH,), jnp.int32), pltpu.VMEM((CH,), jnp.int32),
            pltpu.VMEM((CH, d1), jnp.float32),
            pltpu.VMEM((CH, d1), jnp.float32),
            pltpu.VMEM((CH, d2), jnp.float32),
            pltpu.VMEM((CH, d2), jnp.float32),
            pltpu.SemaphoreType.DMA, pltpu.SemaphoreType.DMA,
            pltpu.SemaphoreType.DMA, pltpu.SemaphoreType.DMA,
        ],
    )
    def gath(t1_hbm, i1_hbm, t2_hbm, i2_hbm, o1_hbm, o2_hbm,
             ia0, ia1, ib0, ib1, ra0, ra1, rb0, rb1, sa0, sa1, sb0, sb1):
        ia, ib = (ia0, ia1), (ib0, ib1)
        ra, rb = (ra0, ra1), (rb0, rb1)
        sa, sb = (sa0, sa1), (sb0, sb1)
        wid = lax.axis_index("s") * 2 + lax.axis_index("c")
        base = wid * epw

        def load_start(buf, off):
            pltpu.sync_copy(i1_hbm.at[pl.ds(off, CH)], ia[buf])
            pltpu.sync_copy(i2_hbm.at[pl.ds(off, CH)], ib[buf])
            pltpu.async_copy(t1_hbm.at[ia[buf]], ra[buf], sa[buf])
            pltpu.async_copy(t2_hbm.at[ib[buf]], rb[buf], sb[buf])

        def wait_store(buf, off):
            pltpu.make_async_copy(t1_hbm.at[ia[buf]], ra[buf], sa[buf]).wait()
            pltpu.make_async_copy(t2_hbm.at[ib[buf]], rb[buf], sb[buf]).wait()
            pltpu.sync_copy(ra[buf], o1_hbm.at[pl.ds(off, CH)])
            pltpu.sync_copy(rb[buf], o2_hbm.at[pl.ds(off, CH)])

        load_start(0, base)

        @pl.loop(0, 2 * ((nch - 1) // 2), step=2)
        def _pair(ci):
            for b in range(2):
                off = base + (ci + b) * CH
                load_start(1 - b, off + CH)
                wait_store(b, off)

        if nch % 2 == 1:
            wait_store(0, base + (nch - 1) * CH)
        else:
            load_start(1, base + (nch - 1) * CH)
            wait_store(0, base + (nch - 2) * CH)
            wait_store(1, base + (nch - 1) * CH)

    return gath


_gather_A = _make_gather2(H, 2 * H, EA, 120)
_gather_B = _make_gather2(H, 2 * H, EB, 80)
_gather0_A = _make_gather2(2 * H, 2 * H, EA, 120)
_gather0_B = _make_gather2(2 * H, 2 * H, EB, 80)


def _make_scatter(ne, ch):
    CH2 = ch
    epw = ne // NW
    assert epw % CH2 == 0 and epw % 8 == 0 and CH2 % 8 == 0 and CH2 <= 128
    nch = epw // CH2

    @functools.partial(
        pl.kernel,
        out_type=_f32((2, NP, H)),
        mesh=_MESH,
        scratch_types=[
            pltpu.VMEM((CH2,), jnp.int32), pltpu.VMEM((CH2,), jnp.int32),
            pltpu.VMEM((CH2, H), jnp.float32),
            pltpu.VMEM((CH2, H), jnp.float32),
            pltpu.VMEM_SHARED((NP, H), jnp.float32),
            pltpu.SemaphoreType.DMA, pltpu.SemaphoreType.DMA,
        ],
    )
    def scat(msg_hbm, dst_hbm, zer_hbm, out_hbm,
             i0, i1, r0, r1, acc, s0, s1):
        c = lax.axis_index("c")
        s = lax.axis_index("s")
        wid = s * 2 + c
        idx, rows, sem = (i0, i1), (r0, r1), (s0, s1)
        pltpu.sync_copy(zer_hbm, acc.at[pl.ds(s * RPT, RPT)])
        plsc.subcore_barrier()
        base = wid * epw

        def load_start(buf, off):
            pltpu.sync_copy(dst_hbm.at[pl.ds(off, CH2)], idx[buf])
            pltpu.async_copy(msg_hbm.at[pl.ds(off, CH2)], rows[buf],
                             sem[buf])

        def wait_add(buf, off):
            pltpu.make_async_copy(msg_hbm.at[pl.ds(off, CH2)], rows[buf],
                                  sem[buf]).wait()
            pltpu.sync_copy(rows[buf], acc.at[idx[buf]], add=True)

        load_start(0, base)

        @pl.loop(0, 2 * ((nch - 1) // 2), step=2)
        def _pair(ci):
            for b in range(2):
                off = base + (ci + b) * CH2
                load_start(1 - b, off + CH2)
                wait_add(b, off)

        if nch % 2 == 1:
            wait_add(0, base + (nch - 1) * CH2)
        else:
            load_start(1, base + (nch - 1) * CH2)
            wait_add(0, base + (nch - 2) * CH2)
            wait_add(1, base + (nch - 1) * CH2)
        plsc.subcore_barrier()
        pltpu.sync_copy(acc.at[pl.ds(s * RPT, RPT)],
                        out_hbm.at[c, pl.ds(s * RPT, RPT)])

    return scat


_scatter_A = _make_scatter(EA, 120)
_scatter_B = _make_scatter(EB, 80)


def kernel(z, pos, edge_index, batch, params):
    src = edge_index[0].astype(jnp.int32)
    dst = edge_index[1].astype(jnp.int32)
    z2 = z.astype(jnp.int32).reshape(N, 1)
    batch2 = batch.astype(jnp.int32).reshape(N // TILE_N, 1, TILE_N)
    pospad = jnp.pad(pos.astype(jnp.float32), ((0, 0), (0, H - 3)))
    zer = jnp.zeros((RPT, H), jnp.float32)

    stop = CUTOFF - CUTOFF / NG
    offs = jnp.pad(jnp.linspace(0.0, stop, NG, dtype=jnp.float32),
                   (0, NGP - NG)).reshape(1, NGP)

    emb = jnp.pad(params['embedding'], ((0, NUM_EMB_P - NUM_EMB), (0, 0)))

    prep = []
    for t, p in enumerate(params['interactions']):
        w1 = p['eu1_W']
        wiT = w1[:, :H].T
        wjT = w1[:, H:2 * H].T
        weT = w1[:, 2 * H:].T
        if t == 0:
            weT = jnp.pad(weT, ((0, NGP - NG), (0, 0)))
        d = dict(
            wiT=wiT, b1=p['eu1_b'].reshape(1, 2 * H), wjT=wjT,
            cfT=p['cf_W'].T,
            weT=weT.astype(jnp.bfloat16),
            eu2T=p['eu2_W'].T.astype(jnp.bfloat16),
            eu2b=p['eu2_b'].reshape(1, H),
            f1T=p['f1_W'].T.astype(jnp.bfloat16),
            f1b=p['f1_b'].reshape(1, H),
            f2T=p['f2_W'].T.astype(jnp.bfloat16),
            f2b=p['f2_b'].reshape(1, H),
            sm1T=p['sm1_W'].T, sm1b=p['sm1_b'].reshape(1, H),
            sm2T=p['sm2_W'].T, sm2b=p['sm2_b'].reshape(1, H),
        )
        if t == NI - 1:
            d['wcT'] = (p['eu2_W'].T @ p['f1_W'].T).astype(jnp.bfloat16)
            d['bc'] = (p['eu2_b'] @ p['f1_W'].T
                       + p['f1_b']).reshape(1, H)
        prep.append(d)

    p0, p1, p2 = prep

    srcA, srcB = src[:EA], src[EA:]
    dstA, dstB = dst[:EA], dst[EA:]

    h0, pd, ps = _init_call(z2, pospad, emb, p0['wiT'], p0['b1'], p0['wjT'],
                            p0['cfT'])

    gdA, gsA = _gather0_A(pd, dstA, ps, srcA)
    gdB, gsB = _gather0_B(pd, dstB, ps, srcB)
    ea = (offs, p0['weT'], p0['eu2T'], p0['eu2b'], p0['f1T'], p0['f1b'],
          p0['f2T'], p0['f2b'])
    eattrA, msgA = _edge0_A(gdA, gsA, *ea)
    eattrB, msgB = _edge0_B(gdB, gsB, *ea)
    mpA = _scatter_A(msgA, dstA, zer)
    mpB = _scatter_B(msgB, dstB, zer)
    h1, pd, ps = _node_call(mpA, mpB, h0, p0['sm1T'], p0['sm1b'],
                            p0['sm2T'], p0['sm2b'], p1['wiT'], p1['b1'],
                            p1['wjT'], p1['cfT'])

    gdA, gsA = _gather_A(pd, dstA, ps, srcA)
    gdB, gsB = _gather_B(pd, dstB, ps, srcB)
    ea = (p1['weT'], p1['eu2T'], p1['eu2b'], p1['f1T'], p1['f1b'],
          p1['f2T'], p1['f2b'])
    eattrA, msgA = _edge1_A(gdA, gsA, eattrA, *ea)
    eattrB, msgB = _edge1_B(gdB, gsB, eattrB, *ea)
    mpA = _scatter_A(msgA, dstA, zer)
    mpB = _scatter_B(msgB, dstB, zer)
    h2, pd, ps = _node_call(mpA, mpB, h1, p1['sm1T'], p1['sm1b'],
                            p1['sm2T'], p1['sm2b'], p2['wiT'], p2['b1'],
                            p2['wjT'], p2['cfT'])

    gdA, gsA = _gather_A(pd, dstA, ps, srcA)
    gdB, gsB = _gather_B(pd, dstB, ps, srcB)
    ea = (p2['weT'], p2['wcT'], p2['bc'], p2['f2T'], p2['f2b'])
    msgA = _edge2_A(gdA, gsA, eattrA, *ea)
    msgB = _edge2_B(gdB, gsB, eattrB, *ea)
    mpA = _scatter_A(msgA, dstA, zer)
    mpB = _scatter_B(msgB, dstB, zer)

    lin1T = params['lin1_W'].T
    lin2T = params['lin2_W'].T
    out = _final_call(mpA, mpB, h2, p2['sm1T'], p2['sm1b'], p2['sm2T'],
                      p2['sm2b'], lin1T,
                      params['lin1_b'].reshape(1, H // 2), lin2T,
                      params['lin2_b'].reshape(1, 1), batch2)
    return out
